# Initial kernel scaffold; baseline (speedup 1.0000x reference)
import jax
import jax.numpy as jnp
from jax.experimental import pallas as pl


def kernel(x, params):
    raise NotImplementedError("write your pallas kernel here")


# speedup vs baseline: 3.2114x; 3.2114x over previous
import jax
import jax.numpy as jnp
from jax.experimental import pallas as pl

DIM = 768
HEADS = 12
DH = 64
INNER = HEADS * DH
MLP_D = 1536
W_WIN = 2
CBS = 4
SBS = 4
NSEL = 2
SCALE = DH ** -0.5
NEG = -1e9


def _f32(x):
    return x.astype(jnp.float32)


def _k1_body(x_ref, g1_ref, b1_ref, wqkv_ref, wg_ref, bg_ref, qkv_ref, g_ref):
    x = x_ref[...]
    m = jnp.mean(x, -1, keepdims=True)
    v = jnp.mean((x - m) ** 2, -1, keepdims=True)
    xn = (x - m) / jnp.sqrt(v + 1e-5) * g1_ref[...] + b1_ref[...]
    qkv_ref[...] = jnp.dot(xn, wqkv_ref[...], preferred_element_type=jnp.float32)
    g_ref[...] = jax.nn.sigmoid(
        jnp.dot(xn, wg_ref[...], preferred_element_type=jnp.float32) + bg_ref[...])


def _k1(x2, ln_g, ln_b, wqkv, wg, bg, tr=512):
    r = x2.shape[0]
    return pl.pallas_call(
        _k1_body,
        grid=(r // tr,),
        in_specs=[
            pl.BlockSpec((tr, DIM), lambda i: (i, 0)),
            pl.BlockSpec((1, DIM), lambda i: (0, 0)),
            pl.BlockSpec((1, DIM), lambda i: (0, 0)),
            pl.BlockSpec((DIM, 3 * INNER), lambda i: (0, 0)),
            pl.BlockSpec((DIM, 3 * HEADS), lambda i: (0, 0)),
            pl.BlockSpec((1, 3 * HEADS), lambda i: (0, 0)),
        ],
        out_specs=[
            pl.BlockSpec((tr, 3 * INNER), lambda i: (i, 0)),
            pl.BlockSpec((tr, 3 * HEADS), lambda i: (i, 0)),
        ],
        out_shape=[
            jax.ShapeDtypeStruct((r, 3 * INNER), jnp.float32),
            jax.ShapeDtypeStruct((r, 3 * HEADS), jnp.float32),
        ],
    )(x2, ln_g[None], ln_b[None], wqkv, wg, bg[None])


def _k2_body(kf_ref, vf_ref, wkc_ref, wvc_ref, bk_ref, bv_ref,
             kp_ref, vp_ref, ck_ref, cv_ref):
    wkc = wkc_ref[...]
    wvc = wvc_ref[...]
    bk = jnp.dot(kp_ref[...], wkc, preferred_element_type=jnp.float32) + bk_ref[...]
    bv = jnp.dot(vp_ref[...], wvc, preferred_element_type=jnp.float32) + bv_ref[...]
    ck_ref[...] = jnp.dot(kf_ref[...], wkc, preferred_element_type=jnp.float32) + bk
    cv_ref[...] = jnp.dot(vf_ref[...], wvc, preferred_element_type=jnp.float32) + bv


def _k2(kf, vf, wkc, wvc, bkc, bvc, kpos, vpos, tr=1024):
    r = kf.shape[0]
    cd = CBS * DH
    return pl.pallas_call(
        _k2_body,
        grid=(r // tr,),
        in_specs=[
            pl.BlockSpec((tr, cd), lambda i: (i, 0)),
            pl.BlockSpec((tr, cd), lambda i: (i, 0)),
            pl.BlockSpec((cd, DH), lambda i: (0, 0)),
            pl.BlockSpec((cd, DH), lambda i: (0, 0)),
            pl.BlockSpec((1, DH), lambda i: (0, 0)),
            pl.BlockSpec((1, DH), lambda i: (0, 0)),
            pl.BlockSpec((1, cd), lambda i: (0, 0)),
            pl.BlockSpec((1, cd), lambda i: (0, 0)),
        ],
        out_specs=[
            pl.BlockSpec((tr, DH), lambda i: (i, 0)),
            pl.BlockSpec((tr, DH), lambda i: (i, 0)),
        ],
        out_shape=[
            jax.ShapeDtypeStruct((r, DH), jnp.float32),
            jax.ShapeDtypeStruct((r, DH), jnp.float32),
        ],
    )(kf, vf, wkc, wvc, bkc[None], bvc[None],
      kpos.reshape(1, cd), vpos.reshape(1, cd))


def _k3_body(q_ref, ck_ref, cv_ref, mk_ref, mv_ref, cout_ref, imp_ref,
             *, tq, nb):
    h = pl.program_id(2)
    qt = pl.program_id(1)
    q = q_ref[0, 0]
    ck = ck_ref[0, 0]
    cv = cv_ref[0, 0]
    s = jax.lax.dot_general(q, ck, (((1,), (1,)), ((), ())),
                            preferred_element_type=jnp.float32) * SCALE
    ivec = qt * tq + jax.lax.broadcasted_iota(jnp.int32, (tq, nb), 0)
    mvec = jax.lax.broadcasted_iota(jnp.int32, (tq, nb), 1)
    s = jnp.where((CBS * mvec + (CBS - 1)) <= ivec, s, NEG)
    smem = jnp.sum(q * mk_ref[0], -1, keepdims=True) * SCALE
    mx = jnp.maximum(jnp.max(s, -1, keepdims=True), smem)
    eb = jnp.exp(s - mx)
    em = jnp.exp(smem - mx)
    den = jnp.sum(eb, -1, keepdims=True) + em
    cout_ref[0, 0] = (jnp.dot(eb, cv, preferred_element_type=jnp.float32)
                      + em * mv_ref[0]) / den
    impc = eb / den * (1.0 / HEADS)

    @pl.when(h == 0)
    def _():
        imp_ref[0] = impc

    @pl.when(h != 0)
    def _():
        imp_ref[0] = imp_ref[0] + impc


def _k3(q, ck, cv, mem_k, mem_v, tq=512):
    b, _, n, _ = q.shape
    nb = ck.shape[2]

    def body(*refs):
        _k3_body(*refs, tq=tq, nb=nb)

    return pl.pallas_call(
        body,
        grid=(b, n // tq, HEADS),
        in_specs=[
            pl.BlockSpec((1, 1, tq, DH), lambda bi, qi, hi: (bi, hi, qi, 0)),
            pl.BlockSpec((1, 1, nb, DH), lambda bi, qi, hi: (bi, hi, 0, 0)),
            pl.BlockSpec((1, 1, nb, DH), lambda bi, qi, hi: (bi, hi, 0, 0)),
            pl.BlockSpec((1, 1, DH), lambda bi, qi, hi: (hi, 0, 0)),
            pl.BlockSpec((1, 1, DH), lambda bi, qi, hi: (hi, 0, 0)),
        ],
        out_specs=[
            pl.BlockSpec((1, 1, tq, DH), lambda bi, qi, hi: (bi, hi, qi, 0)),
            pl.BlockSpec((1, tq, nb), lambda bi, qi, hi: (bi, qi, 0)),
        ],
        out_shape=[
            jax.ShapeDtypeStruct((b, HEADS, n, DH), jnp.float32),
            jax.ShapeDtypeStruct((b, n, nb), jnp.float32),
        ],
    )(q, ck, cv, mem_k[:, None], mem_v[:, None])


def _k4_body(imp_ref, sel_ref, *, tq, nb):
    v = imp_ref[0]
    j = jax.lax.broadcasted_iota(jnp.int32, (tq, nb), 1)
    m1 = jnp.max(v, -1, keepdims=True)
    i1 = jnp.min(jnp.where(v == m1, j, nb), -1, keepdims=True)
    v2 = jnp.where(j == i1, -jnp.inf, v)
    m2 = jnp.max(v2, -1, keepdims=True)
    i2 = jnp.min(jnp.where(v2 == m2, j, nb), -1, keepdims=True)
    sel_ref[0] = jnp.concatenate([i1, i2], axis=-1)


def _k4(imp, tq=512):
    b, n, nb = imp.shape

    def body(*refs):
        _k4_body(*refs, tq=tq, nb=nb)

    return pl.pallas_call(
        body,
        grid=(b, n // tq),
        in_specs=[pl.BlockSpec((1, tq, nb), lambda bi, qi: (bi, qi, 0))],
        out_specs=pl.BlockSpec((1, tq, NSEL), lambda bi, qi: (bi, qi, 0)),
        out_shape=jax.ShapeDtypeStruct((b, n, NSEL), jnp.int32),
    )(imp)


def _k5_body(q_ref, k_ref, v_ref, cout_ref, sel_ref, g_ref, out_ref,
             *, tq, n):
    qt = pl.program_id(2)
    qs = qt * tq
    q = q_ref[0, 0]
    k = k_ref[0, 0]
    v = v_ref[0, 0]
    s = jax.lax.dot_general(q, k, (((1,), (1,)), ((), ())),
                            preferred_element_type=jnp.float32) * SCALE
    i2 = qs + jax.lax.broadcasted_iota(jnp.int32, (tq, n), 0)
    j2 = jax.lax.broadcasted_iota(jnp.int32, (tq, n), 1)
    jblk = j2 // SBS
    sel = sel_ref[0]
    sel0 = sel[:, 0:1]
    sel1 = sel[:, 1:2]
    icol = qs + jax.lax.broadcasted_iota(jnp.int32, (tq, 1), 0)
    own = icol // SBS
    w = ((jblk == sel0).astype(jnp.float32)
         + (jblk == sel1).astype(jnp.float32)
         + (jblk == own).astype(jnp.float32))
    causal = j2 <= i2
    s_eff = jnp.where(jnp.logical_and(causal, w > 0), s, NEG)
    mx = jnp.max(s_eff, -1, keepdims=True)
    e = jnp.where(causal, w, 0.0) * jnp.exp(s_eff - mx)
    den = jnp.sum(e, -1, keepdims=True)
    sout = jnp.dot(e, v, preferred_element_type=jnp.float32) / den
    kt = k_ref[0, 0, pl.ds(qs, tq), :]
    vt = v_ref[0, 0, pl.ds(qs, tq), :]
    pstart = jnp.maximum(qs - 1, 0)
    kp_row = k_ref[0, 0, pl.ds(pstart, 1), :]
    vp_row = v_ref[0, 0, pl.ds(pstart, 1), :]
    kprev = jnp.concatenate([kp_row, kt[:-1]], axis=0)
    vprev = jnp.concatenate([vp_row, vt[:-1]], axis=0)
    s1 = jnp.sum(q * kt, -1, keepdims=True) * SCALE
    s0 = jnp.sum(q * kprev, -1, keepdims=True) * SCALE
    s0 = jnp.where(icol > 0, s0, NEG)
    mw = jnp.maximum(s0, s1)
    e0 = jnp.exp(s0 - mw)
    e1 = jnp.exp(s1 - mw)
    wout = (e0 * vprev + e1 * vt) / (e0 + e1)
    g = g_ref[0, 0]
    out_ref[0, 0] = (g[:, 0:1] * cout_ref[0, 0] + g[:, 1:2] * sout
                     + g[:, 2:3] * wout)


def _k5(q, k, v, cout, sel, g3, tq=512):
    b, _, n, _ = q.shape

    def body(*refs):
        _k5_body(*refs, tq=tq, n=n)

    return pl.pallas_call(
        body,
        grid=(b, HEADS, n // tq),
        in_specs=[
            pl.BlockSpec((1, 1, tq, DH), lambda bi, hi, qi: (bi, hi, qi, 0)),
            pl.BlockSpec((1, 1, n, DH), lambda bi, hi, qi: (bi, hi, 0, 0)),
            pl.BlockSpec((1, 1, n, DH), lambda bi, hi, qi: (bi, hi, 0, 0)),
            pl.BlockSpec((1, 1, tq, DH), lambda bi, hi, qi: (bi, hi, qi, 0)),
            pl.BlockSpec((1, tq, NSEL), lambda bi, hi, qi: (bi, qi, 0)),
            pl.BlockSpec((1, 1, tq, 3), lambda bi, hi, qi: (bi, hi, qi, 0)),
        ],
        out_specs=pl.BlockSpec((1, 1, tq, DH),
                               lambda bi, hi, qi: (bi, hi, qi, 0)),
        out_shape=jax.ShapeDtypeStruct((b, HEADS, n, DH), jnp.float32),
    )(q, k, v, cout, sel, g3)


def _k6_body(a_ref, w_ref, r_ref, o_ref):
    o_ref[...] = (jnp.dot(a_ref[...], w_ref[...],
                          preferred_element_type=jnp.float32) + r_ref[...])


def _k6(a, w, res, tr=512):
    r, d_in = a.shape
    d_out = w.shape[1]
    return pl.pallas_call(
        _k6_body,
        grid=(r // tr,),
        in_specs=[
            pl.BlockSpec((tr, d_in), lambda i: (i, 0)),
            pl.BlockSpec((d_in, d_out), lambda i: (0, 0)),
            pl.BlockSpec((tr, d_out), lambda i: (i, 0)),
        ],
        out_specs=pl.BlockSpec((tr, d_out), lambda i: (i, 0)),
        out_shape=jax.ShapeDtypeStruct((r, d_out), jnp.float32),
    )(a, w, res)


def _k7_body(x_ref, g2_ref, b2_ref, w_ref, b_ref, o_ref):
    x = x_ref[...]
    m = jnp.mean(x, -1, keepdims=True)
    v = jnp.mean((x - m) ** 2, -1, keepdims=True)
    xn = (x - m) / jnp.sqrt(v + 1e-5) * g2_ref[...] + b2_ref[...]
    z = jnp.dot(xn, w_ref[...], preferred_element_type=jnp.float32) + b_ref[...]
    o_ref[...] = jnp.where(z >= 0, z, 0.01 * z)


def _k7(x2, ln_g, ln_b, w1, b1, tr=512):
    r, d_in = x2.shape
    d_out = w1.shape[1]
    return pl.pallas_call(
        _k7_body,
        grid=(r // tr,),
        in_specs=[
            pl.BlockSpec((tr, d_in), lambda i: (i, 0)),
            pl.BlockSpec((1, d_in), lambda i: (0, 0)),
            pl.BlockSpec((1, d_in), lambda i: (0, 0)),
            pl.BlockSpec((d_in, d_out), lambda i: (0, 0)),
            pl.BlockSpec((1, d_out), lambda i: (0, 0)),
        ],
        out_specs=pl.BlockSpec((tr, d_out), lambda i: (i, 0)),
        out_shape=jax.ShapeDtypeStruct((r, d_out), jnp.float32),
    )(x2, ln_g[None], ln_b[None], w1, b1[None])


def _k8_body(a_ref, w_ref, b_ref, r_ref, o_ref):
    o_ref[...] = (jnp.dot(a_ref[...], w_ref[...],
                          preferred_element_type=jnp.float32)
                  + b_ref[...] + r_ref[...])


def _k8(a, w, bias, res, tr=512):
    r, d_in = a.shape
    d_out = w.shape[1]
    return pl.pallas_call(
        _k8_body,
        grid=(r // tr,),
        in_specs=[
            pl.BlockSpec((tr, d_in), lambda i: (i, 0)),
            pl.BlockSpec((d_in, d_out), lambda i: (0, 0)),
            pl.BlockSpec((1, d_out), lambda i: (0, 0)),
            pl.BlockSpec((tr, d_out), lambda i: (i, 0)),
        ],
        out_specs=pl.BlockSpec((tr, d_out), lambda i: (i, 0)),
        out_shape=jax.ShapeDtypeStruct((r, d_out), jnp.float32),
    )(a, w, bias[None], res)


def _layer(x, p):
    b, n, _ = x.shape
    r = b * n
    nb = n // CBS
    x2 = x.reshape(r, DIM)
    qkv, g36 = _k1(x2, p['ln1_g'], p['ln1_b'], p['Wqkv'], p['Wg'], p['bg'])
    qkv4 = qkv.reshape(b, n, 3, HEADS, DH)
    q = qkv4[:, :, 0].transpose(0, 2, 1, 3)
    k = qkv4[:, :, 1].transpose(0, 2, 1, 3)
    v = qkv4[:, :, 2].transpose(0, 2, 1, 3)
    g3 = g36.reshape(b, n, 3, HEADS).transpose(0, 3, 1, 2)
    kf = k.reshape(b * HEADS * nb, CBS * DH)
    vf = v.reshape(b * HEADS * nb, CBS * DH)
    ck, cv = _k2(kf, vf, p['Wkc'], p['Wvc'], p['bkc'], p['bvc'],
                 p['k_pos'], p['v_pos'])
    ck = ck.reshape(b, HEADS, nb, DH)
    cv = cv.reshape(b, HEADS, nb, DH)
    cout, imp = _k3(q, ck, cv, p['mem_k'], p['mem_v'])
    sel = _k4(imp)
    comb = _k5(q, k, v, cout, sel, g3)
    comb2 = comb.transpose(0, 2, 1, 3).reshape(r, INNER)
    y = _k6(comb2, p['Wo'], x2)
    h1 = _k7(y, p['ln2_g'], p['ln2_b'], p['W1'], p['b1'])
    out = _k8(h1, p['W2'], p['b2'], y)
    return out.reshape(b, n, DIM)


def kernel(x, params):
    for p in params:
        x = _layer(x, p)
    return x


# speedup vs baseline: 5.3138x; 1.6547x over previous
import jax
import jax.numpy as jnp
from jax.experimental import pallas as pl
from jax.experimental.pallas import tpu as pltpu

DIM = 768
HEADS = 12
NPAIR = HEADS // 2
DH = 64
INNER = HEADS * DH
MLP_D = 1536
W_WIN = 2
CBS = 4
SBS = 4
NSEL = 2
SCALE = DH ** -0.5
NEG = -1e9


def _k1_body(x_ref, g1_ref, b1_ref, wqkv_ref, wg_ref, bg_ref, qkv_ref, g_ref):
    x = x_ref[...]
    m = jnp.mean(x, -1, keepdims=True)
    v = jnp.mean((x - m) ** 2, -1, keepdims=True)
    xn = (x - m) / jnp.sqrt(v + 1e-5) * g1_ref[...] + b1_ref[...]
    qkv_ref[...] = jnp.dot(xn, wqkv_ref[...], preferred_element_type=jnp.float32)
    g_ref[...] = jax.nn.sigmoid(
        jnp.dot(xn, wg_ref[...], preferred_element_type=jnp.float32) + bg_ref[...])


def _k1(x2, ln_g, ln_b, wqkv, wg, bg, tr=512):
    r = x2.shape[0]
    return pl.pallas_call(
        _k1_body,
        grid=(r // tr,),
        in_specs=[
            pl.BlockSpec((tr, DIM), lambda i: (i, 0)),
            pl.BlockSpec((1, DIM), lambda i: (0, 0)),
            pl.BlockSpec((1, DIM), lambda i: (0, 0)),
            pl.BlockSpec((DIM, 3 * INNER), lambda i: (0, 0)),
            pl.BlockSpec((DIM, 3 * HEADS), lambda i: (0, 0)),
            pl.BlockSpec((1, 3 * HEADS), lambda i: (0, 0)),
        ],
        out_specs=[
            pl.BlockSpec((tr, 3 * INNER), lambda i: (i, 0)),
            pl.BlockSpec((tr, 3 * HEADS), lambda i: (i, 0)),
        ],
        out_shape=[
            jax.ShapeDtypeStruct((r, 3 * INNER), jnp.float32),
            jax.ShapeDtypeStruct((r, 3 * HEADS), jnp.float32),
        ],
    )(x2, ln_g[None], ln_b[None], wqkv, wg, bg[None])


def _k2_body(kf_ref, vf_ref, wkc_ref, wvc_ref, bk_ref, bv_ref,
             kp_ref, vp_ref, ck_ref, cv_ref, *, nb):
    wkc = wkc_ref[...]
    wvc = wvc_ref[...]
    bk = jnp.dot(kp_ref[...], wkc, preferred_element_type=jnp.float32) + bk_ref[...]
    bv = jnp.dot(vp_ref[...], wvc, preferred_element_type=jnp.float32) + bv_ref[...]
    ck = jnp.dot(kf_ref[...], wkc, preferred_element_type=jnp.float32) + bk
    cv = jnp.dot(vf_ref[...], wvc, preferred_element_type=jnp.float32) + bv
    ck_ref[0] = jnp.concatenate([ck[:nb], ck[nb:]], axis=-1)
    cv_ref[0] = jnp.concatenate([cv[:nb], cv[nb:]], axis=-1)


def _k2(kf, vf, wkc, wvc, bkc, bvc, kpos, vpos, b, nb):
    cd = CBS * DH

    def body(*refs):
        _k2_body(*refs, nb=nb)

    return pl.pallas_call(
        body,
        grid=(b, NPAIR),
        in_specs=[
            pl.BlockSpec((2 * nb, cd), lambda bi, j: (bi * NPAIR + j, 0)),
            pl.BlockSpec((2 * nb, cd), lambda bi, j: (bi * NPAIR + j, 0)),
            pl.BlockSpec((cd, DH), lambda bi, j: (0, 0)),
            pl.BlockSpec((cd, DH), lambda bi, j: (0, 0)),
            pl.BlockSpec((1, DH), lambda bi, j: (0, 0)),
            pl.BlockSpec((1, DH), lambda bi, j: (0, 0)),
            pl.BlockSpec((1, cd), lambda bi, j: (0, 0)),
            pl.BlockSpec((1, cd), lambda bi, j: (0, 0)),
        ],
        out_specs=[
            pl.BlockSpec((1, nb, 2 * DH), lambda bi, j: (bi, 0, j)),
            pl.BlockSpec((1, nb, 2 * DH), lambda bi, j: (bi, 0, j)),
        ],
        out_shape=[
            jax.ShapeDtypeStruct((b, nb, INNER), jnp.float32),
            jax.ShapeDtypeStruct((b, nb, INNER), jnp.float32),
        ],
    )(kf, vf, wkc, wvc, bkc[None], bvc[None],
      kpos.reshape(1, cd), vpos.reshape(1, cd))


def _k3_body(q_ref, ck_ref, cv_ref, mk_ref, mv_ref, cout_ref, sel_ref,
             imp_ref, *, tq, nb, qt):
    j = pl.program_id(1)
    q2 = q_ref[0]
    ck2 = ck_ref[0]
    cv2 = cv_ref[0]
    mk2 = mk_ref[0]
    mv2 = mv_ref[0]
    ivec = qt * tq + jax.lax.broadcasted_iota(jnp.int32, (tq, nb), 0)
    mvec = jax.lax.broadcasted_iota(jnp.int32, (tq, nb), 1)
    bias = jnp.where((CBS * mvec + (CBS - 1)) <= ivec, 0.0, NEG)
    couts = []
    imps = []
    for hh in range(2):
        q = q2[:, hh * DH:(hh + 1) * DH]
        ck = ck2[:, hh * DH:(hh + 1) * DH]
        cv = cv2[:, hh * DH:(hh + 1) * DH]
        s = jax.lax.dot_general(q, ck, (((1,), (1,)), ((), ())),
                                preferred_element_type=jnp.float32) * SCALE
        s = s + bias
        smem = jnp.sum(q * mk2[:, hh * DH:(hh + 1) * DH], -1,
                       keepdims=True) * SCALE
        mx = jnp.maximum(jnp.max(s, -1, keepdims=True), smem)
        eb = jnp.exp(s - mx)
        em = jnp.exp(smem - mx)
        den = jnp.sum(eb, -1, keepdims=True) + em
        couts.append((jnp.dot(eb, cv, preferred_element_type=jnp.float32)
                      + em * mv2[:, hh * DH:(hh + 1) * DH]) / den)
        imps.append(eb / den)
    cout_ref[0] = jnp.concatenate(couts, axis=-1)
    impc = (imps[0] + imps[1]) * (1.0 / HEADS)

    @pl.when(j == 0)
    def _():
        imp_ref[...] = impc

    @pl.when(j != 0)
    def _():
        imp_ref[...] = imp_ref[...] + impc

    @pl.when(j == NPAIR - 1)
    def _():
        vimp = imp_ref[...]
        jj = jax.lax.broadcasted_iota(jnp.int32, (tq, nb), 1)
        m1 = jnp.max(vimp, -1, keepdims=True)
        i1 = jnp.min(jnp.where(vimp == m1, jj, nb), -1, keepdims=True)
        v2 = jnp.where(jj == i1, -jnp.inf, vimp)
        m2 = jnp.max(v2, -1, keepdims=True)
        i2 = jnp.min(jnp.where(v2 == m2, jj, nb), -1, keepdims=True)
        sel_ref[0] = jnp.concatenate([i1, i2], axis=-1)


def _k3(qkv3, ck, cv, mem_kp, mem_vp, qt, tq=512):
    b = qkv3.shape[0]
    nb = (qt + 1) * tq // CBS

    def body(*refs):
        _k3_body(*refs, tq=tq, nb=nb, qt=qt)

    return pl.pallas_call(
        body,
        grid=(b, NPAIR),
        in_specs=[
            pl.BlockSpec((1, tq, 2 * DH), lambda bi, j: (bi, qt, j)),
            pl.BlockSpec((1, nb, 2 * DH), lambda bi, j: (bi, 0, j)),
            pl.BlockSpec((1, nb, 2 * DH), lambda bi, j: (bi, 0, j)),
            pl.BlockSpec((1, 1, 2 * DH), lambda bi, j: (j, 0, 0)),
            pl.BlockSpec((1, 1, 2 * DH), lambda bi, j: (j, 0, 0)),
        ],
        out_specs=[
            pl.BlockSpec((1, tq, 2 * DH), lambda bi, j: (bi, 0, j)),
            pl.BlockSpec((1, tq, NSEL), lambda bi, j: (bi, 0, 0)),
        ],
        out_shape=[
            jax.ShapeDtypeStruct((b, tq, INNER), jnp.float32),
            jax.ShapeDtypeStruct((b, tq, NSEL), jnp.int32),
        ],
        scratch_shapes=[pltpu.VMEM((tq, nb), jnp.float32)],
    )(qkv3, ck, cv, mem_kp, mem_vp)


def _k5_body(q_ref, k_ref, v_ref, cout_ref, sel_ref, g_ref, out_ref,
             wc_ref, bias_ref, *, tq, nw, qt):
    j = pl.program_id(1)
    qs = qt * tq

    @pl.when(j == 0)
    def _():
        i2 = qs + jax.lax.broadcasted_iota(jnp.int32, (tq, nw), 0)
        j2 = jax.lax.broadcasted_iota(jnp.int32, (tq, nw), 1)
        jblk = j2 // SBS
        sel = sel_ref[0]
        sel0 = sel[:, 0:1]
        sel1 = sel[:, 1:2]
        icol0 = qs + jax.lax.broadcasted_iota(jnp.int32, (tq, 1), 0)
        own = icol0 // SBS
        w = ((jblk == sel0).astype(jnp.float32)
             + (jblk == sel1).astype(jnp.float32)
             + (jblk == own).astype(jnp.float32))
        causal = j2 <= i2
        wc = jnp.where(causal, w, 0.0)
        wc_ref[...] = wc
        bias_ref[...] = jnp.where(wc > 0.0, 0.0, NEG)

    q2 = q_ref[0]
    k2 = k_ref[0]
    v2 = v_ref[0]
    icol = qs + jax.lax.broadcasted_iota(jnp.int32, (tq, 1), 0)
    kti = k_ref[0, pl.ds(qs, tq), :]
    vti = v_ref[0, pl.ds(qs, tq), :]
    pstart = jnp.maximum(qs - 1, 0)
    kpi = jnp.concatenate([k_ref[0, pl.ds(pstart, 1), :], kti[:-1]], axis=0)
    vpi = jnp.concatenate([v_ref[0, pl.ds(pstart, 1), :], vti[:-1]], axis=0)
    wcm = wc_ref[...]
    sbias = bias_ref[...]
    outs = []
    for hh in range(2):
        sl = slice(hh * DH, (hh + 1) * DH)
        q = q2[:, sl]
        k = k2[:, sl]
        v = v2[:, sl]
        s = jax.lax.dot_general(q.astype(jnp.bfloat16), k.astype(jnp.bfloat16),
                                (((1,), (1,)), ((), ())),
                                preferred_element_type=jnp.float32) * SCALE
        s_eff = s + sbias
        mx = jnp.max(s_eff, -1, keepdims=True)
        e = wcm * jnp.exp(s_eff - mx)
        den = jnp.sum(e, -1, keepdims=True)
        sout = jnp.dot(e.astype(jnp.bfloat16), v.astype(jnp.bfloat16),
                       preferred_element_type=jnp.float32) / den
        kt = kti[:, sl]
        vt = vti[:, sl]
        kprev = kpi[:, sl]
        vprev = vpi[:, sl]
        s1 = jnp.sum(q * kt, -1, keepdims=True) * SCALE
        s0 = jnp.sum(q * kprev, -1, keepdims=True) * SCALE
        s0 = jnp.where(icol > 0, s0, NEG)
        mw = jnp.maximum(s0, s1)
        e0 = jnp.exp(s0 - mw)
        e1 = jnp.exp(s1 - mw)
        wout = (e0 * vprev + e1 * vt) / (e0 + e1)
        g = g_ref[0, hh]
        outs.append(g[:, 0:1] * cout_ref[0][:, sl] + g[:, 1:2] * sout
                    + g[:, 2:3] * wout)
    out_ref[0] = jnp.concatenate(outs, axis=-1)


def _k5(qkv3, cout, sel, g3p, qt, tq=512):
    b, n, _ = qkv3.shape
    nw = (qt + 1) * tq

    def body(*refs):
        _k5_body(*refs, tq=tq, nw=nw, qt=qt)

    return pl.pallas_call(
        body,
        grid=(b, NPAIR),
        in_specs=[
            pl.BlockSpec((1, tq, 2 * DH), lambda bi, j: (bi, qt, j)),
            pl.BlockSpec((1, nw, 2 * DH), lambda bi, j: (bi, 0, NPAIR + j)),
            pl.BlockSpec((1, nw, 2 * DH), lambda bi, j: (bi, 0, 2 * NPAIR + j)),
            pl.BlockSpec((1, tq, 2 * DH), lambda bi, j: (bi, 0, j)),
            pl.BlockSpec((1, tq, NSEL), lambda bi, j: (bi, 0, 0)),
            pl.BlockSpec((1, 2, tq, 3), lambda bi, j: (bi, j, qt, 0)),
        ],
        out_specs=pl.BlockSpec((1, tq, 2 * DH), lambda bi, j: (bi, 0, j)),
        out_shape=jax.ShapeDtypeStruct((b, tq, INNER), jnp.float32),
        scratch_shapes=[
            pltpu.VMEM((tq, nw), jnp.float32),
            pltpu.VMEM((tq, nw), jnp.float32),
        ],
    )(qkv3, qkv3, qkv3, cout, sel, g3p)


def _k6_body(a_ref, w_ref, r_ref, o_ref):
    o_ref[...] = (jnp.dot(a_ref[...].astype(jnp.bfloat16), w_ref[...],
                          preferred_element_type=jnp.float32) + r_ref[...])


def _k6(a, w, res, tr=512):
    r, d_in = a.shape
    d_out = w.shape[1]
    return pl.pallas_call(
        _k6_body,
        grid=(r // tr,),
        in_specs=[
            pl.BlockSpec((tr, d_in), lambda i: (i, 0)),
            pl.BlockSpec((d_in, d_out), lambda i: (0, 0)),
            pl.BlockSpec((tr, d_out), lambda i: (i, 0)),
        ],
        out_specs=pl.BlockSpec((tr, d_out), lambda i: (i, 0)),
        out_shape=jax.ShapeDtypeStruct((r, d_out), jnp.float32),
    )(a, w, res)


def _k7_body(x_ref, g2_ref, b2_ref, w_ref, b_ref, o_ref):
    x = x_ref[...]
    m = jnp.mean(x, -1, keepdims=True)
    v = jnp.mean((x - m) ** 2, -1, keepdims=True)
    xn = (x - m) / jnp.sqrt(v + 1e-5) * g2_ref[...] + b2_ref[...]
    z = jnp.dot(xn.astype(jnp.bfloat16), w_ref[...],
                preferred_element_type=jnp.float32) + b_ref[...]
    o_ref[...] = jnp.where(z >= 0, z, 0.01 * z).astype(jnp.bfloat16)


def _k7(x2, ln_g, ln_b, w1, b1, tr=512):
    r, d_in = x2.shape
    d_out = w1.shape[1]
    return pl.pallas_call(
        _k7_body,
        grid=(r // tr,),
        in_specs=[
            pl.BlockSpec((tr, d_in), lambda i: (i, 0)),
            pl.BlockSpec((1, d_in), lambda i: (0, 0)),
            pl.BlockSpec((1, d_in), lambda i: (0, 0)),
            pl.BlockSpec((d_in, d_out), lambda i: (0, 0)),
            pl.BlockSpec((1, d_out), lambda i: (0, 0)),
        ],
        out_specs=pl.BlockSpec((tr, d_out), lambda i: (i, 0)),
        out_shape=jax.ShapeDtypeStruct((r, d_out), jnp.bfloat16),
    )(x2, ln_g[None], ln_b[None], w1, b1[None])


def _k8_body(a_ref, w_ref, b_ref, r_ref, o_ref):
    o_ref[...] = (jnp.dot(a_ref[...], w_ref[...],
                          preferred_element_type=jnp.float32)
                  + b_ref[...] + r_ref[...])


def _k8(a, w, bias, res, tr=512):
    r, d_in = a.shape
    d_out = w.shape[1]
    return pl.pallas_call(
        _k8_body,
        grid=(r // tr,),
        in_specs=[
            pl.BlockSpec((tr, d_in), lambda i: (i, 0)),
            pl.BlockSpec((d_in, d_out), lambda i: (0, 0)),
            pl.BlockSpec((1, d_out), lambda i: (0, 0)),
            pl.BlockSpec((tr, d_out), lambda i: (i, 0)),
        ],
        out_specs=pl.BlockSpec((tr, d_out), lambda i: (i, 0)),
        out_shape=jax.ShapeDtypeStruct((r, d_out), jnp.float32),
    )(a, w, bias[None], res)


def _layer(x, p):
    b, n, _ = x.shape
    r = b * n
    nb = n // CBS
    x2 = x.reshape(r, DIM)
    qkv, g36 = _k1(x2, p['ln1_g'], p['ln1_b'], p['Wqkv'], p['Wg'], p['bg'])
    qkv3 = qkv.reshape(b, n, 3 * INNER)
    k_bh = qkv3[:, :, INNER:2 * INNER].reshape(b, n, HEADS, DH).transpose(0, 2, 1, 3)
    v_bh = qkv3[:, :, 2 * INNER:].reshape(b, n, HEADS, DH).transpose(0, 2, 1, 3)
    kf = k_bh.reshape(b * HEADS * nb, CBS * DH)
    vf = v_bh.reshape(b * HEADS * nb, CBS * DH)
    ck, cv = _k2(kf, vf, p['Wkc'], p['Wvc'], p['bkc'], p['bvc'],
                 p['k_pos'], p['v_pos'], b, nb)
    g3p = g36.reshape(b, n, 3, HEADS).transpose(0, 3, 1, 2)
    mem_kp = p['mem_k'].reshape(NPAIR, 1, 2 * DH)
    mem_vp = p['mem_v'].reshape(NPAIR, 1, 2 * DH)
    tq = 512
    combs = []
    for qt in range(n // tq):
        cout_t, sel_t = _k3(qkv3, ck, cv, mem_kp, mem_vp, qt, tq=tq)
        combs.append(_k5(qkv3, cout_t, sel_t, g3p, qt, tq=tq))
    comb2 = jnp.concatenate(combs, axis=1).reshape(r, INNER)
    y = _k6(comb2, p['Wo'].astype(jnp.bfloat16), x2)
    h1 = _k7(y, p['ln2_g'], p['ln2_b'], p['W1'].astype(jnp.bfloat16), p['b1'])
    out = _k8(h1, p['W2'].astype(jnp.bfloat16), p['b2'], y)
    return out.reshape(b, n, DIM)


def kernel(x, params):
    for p in params:
        x = _layer(x, p)
    return x


# speedup vs baseline: 6.1469x; 1.1568x over previous
import jax
import jax.numpy as jnp
from jax.experimental import pallas as pl
from jax.experimental.pallas import tpu as pltpu

DIM = 768
HEADS = 12
NPAIR = HEADS // 2
DH = 64
INNER = HEADS * DH
MLP_D = 1536
W_WIN = 2
CBS = 4
SBS = 4
NSEL = 2
SCALE = DH ** -0.5
NEG = -1e9


def _k1_body(x_ref, g1_ref, b1_ref, wqkv_ref, wg_ref, bg_ref, qkv_ref, g_ref):
    x = x_ref[...]
    m = jnp.mean(x, -1, keepdims=True)
    v = jnp.mean((x - m) ** 2, -1, keepdims=True)
    xn = (x - m) / jnp.sqrt(v + 1e-5) * g1_ref[...] + b1_ref[...]
    qkv_ref[...] = jnp.dot(xn, wqkv_ref[...], preferred_element_type=jnp.float32)
    g_ref[...] = jax.nn.sigmoid(
        jnp.dot(xn, wg_ref[...], preferred_element_type=jnp.float32) + bg_ref[...])


def _k1(x2, ln_g, ln_b, wqkv, wg, bg, tr=512):
    r = x2.shape[0]
    return pl.pallas_call(
        _k1_body,
        grid=(r // tr,),
        in_specs=[
            pl.BlockSpec((tr, DIM), lambda i: (i, 0)),
            pl.BlockSpec((1, DIM), lambda i: (0, 0)),
            pl.BlockSpec((1, DIM), lambda i: (0, 0)),
            pl.BlockSpec((DIM, 3 * INNER), lambda i: (0, 0)),
            pl.BlockSpec((DIM, 3 * HEADS), lambda i: (0, 0)),
            pl.BlockSpec((1, 3 * HEADS), lambda i: (0, 0)),
        ],
        out_specs=[
            pl.BlockSpec((tr, 3 * INNER), lambda i: (i, 0)),
            pl.BlockSpec((tr, 3 * HEADS), lambda i: (i, 0)),
        ],
        out_shape=[
            jax.ShapeDtypeStruct((r, 3 * INNER), jnp.float32),
            jax.ShapeDtypeStruct((r, 3 * HEADS), jnp.float32),
        ],
    )(x2, ln_g[None], ln_b[None], wqkv, wg, bg[None])


def _k2_body(k0_ref, k1_ref, k2_ref, k3_ref, v0_ref, v1_ref, v2_ref, v3_ref,
             wkc_ref, wvc_ref, bk_ref, bv_ref, kp_ref, vp_ref,
             ck_ref, cv_ref):
    wkc = wkc_ref[...]
    wvc = wvc_ref[...]
    bk = jnp.dot(kp_ref[...], wkc, preferred_element_type=jnp.float32) + bk_ref[...]
    bv = jnp.dot(vp_ref[...], wvc, preferred_element_type=jnp.float32) + bv_ref[...]
    krs = [k0_ref[0], k1_ref[0], k2_ref[0], k3_ref[0]]
    vrs = [v0_ref[0], v1_ref[0], v2_ref[0], v3_ref[0]]
    cks = []
    cvs = []
    for hh in range(2):
        sl = slice(hh * DH, (hh + 1) * DH)
        ck = bk
        cv = bv
        for jr in range(CBS):
            wsl = slice(jr * DH, (jr + 1) * DH)
            ck = ck + jnp.dot(krs[jr][:, sl], wkc[wsl],
                              preferred_element_type=jnp.float32)
            cv = cv + jnp.dot(vrs[jr][:, sl], wvc[wsl],
                              preferred_element_type=jnp.float32)
        cks.append(ck)
        cvs.append(cv)
    ck_ref[0] = jnp.concatenate(cks, axis=-1)
    cv_ref[0] = jnp.concatenate(cvs, axis=-1)


def _k2(qkv3, wkc, wvc, bkc, bvc, kpos, vpos):
    cd = CBS * DH
    b, n, _ = qkv3.shape
    nb = n // CBS
    q4 = qkv3.reshape(b, nb, CBS * 3 * INNER)
    kspec = [pl.BlockSpec((1, nb, 2 * DH),
                          (lambda bi, j, jr_=jr: (bi, 0, 18 * jr_ + NPAIR + j)))
             for jr in range(CBS)]
    vspec = [pl.BlockSpec((1, nb, 2 * DH),
                          (lambda bi, j, jr_=jr: (bi, 0, 18 * jr_ + 2 * NPAIR + j)))
             for jr in range(CBS)]
    return pl.pallas_call(
        _k2_body,
        grid=(b, NPAIR),
        in_specs=kspec + vspec + [
            pl.BlockSpec((cd, DH), lambda bi, j: (0, 0)),
            pl.BlockSpec((cd, DH), lambda bi, j: (0, 0)),
            pl.BlockSpec((1, DH), lambda bi, j: (0, 0)),
            pl.BlockSpec((1, DH), lambda bi, j: (0, 0)),
            pl.BlockSpec((1, cd), lambda bi, j: (0, 0)),
            pl.BlockSpec((1, cd), lambda bi, j: (0, 0)),
        ],
        out_specs=[
            pl.BlockSpec((1, nb, 2 * DH), lambda bi, j: (bi, 0, j)),
            pl.BlockSpec((1, nb, 2 * DH), lambda bi, j: (bi, 0, j)),
        ],
        out_shape=[
            jax.ShapeDtypeStruct((b, nb, INNER), jnp.float32),
            jax.ShapeDtypeStruct((b, nb, INNER), jnp.float32),
        ],
    )(q4, q4, q4, q4, q4, q4, q4, q4, wkc, wvc, bkc[None], bvc[None],
      kpos.reshape(1, cd), vpos.reshape(1, cd))


def _k3_body(q_ref, ck_ref, cv_ref, mk_ref, mv_ref, cout_ref, sel_ref,
             imp_ref, *, tq, nb, qt):
    j = pl.program_id(1)
    q2 = q_ref[0]
    ck2 = ck_ref[0]
    cv2 = cv_ref[0]
    mk2 = mk_ref[0]
    mv2 = mv_ref[0]
    ivec = qt * tq + jax.lax.broadcasted_iota(jnp.int32, (tq, nb), 0)
    mvec = jax.lax.broadcasted_iota(jnp.int32, (tq, nb), 1)
    bias = jnp.where((CBS * mvec + (CBS - 1)) <= ivec, 0.0, NEG)
    couts = []
    imps = []
    for hh in range(2):
        q = q2[:, hh * DH:(hh + 1) * DH]
        ck = ck2[:, hh * DH:(hh + 1) * DH]
        cv = cv2[:, hh * DH:(hh + 1) * DH]
        s = jax.lax.dot_general(q, ck, (((1,), (1,)), ((), ())),
                                preferred_element_type=jnp.float32) * SCALE
        s = s + bias
        smem = jnp.sum(q * mk2[:, hh * DH:(hh + 1) * DH], -1,
                       keepdims=True) * SCALE
        mx = jnp.maximum(jnp.max(s, -1, keepdims=True), smem)
        eb = jnp.exp(s - mx)
        em = jnp.exp(smem - mx)
        den = jnp.sum(eb, -1, keepdims=True) + em
        couts.append((jnp.dot(eb, cv, preferred_element_type=jnp.float32)
                      + em * mv2[:, hh * DH:(hh + 1) * DH]) / den)
        imps.append(eb / den)
    cout_ref[0] = jnp.concatenate(couts, axis=-1)
    impc = (imps[0] + imps[1]) * (1.0 / HEADS)

    @pl.when(j == 0)
    def _():
        imp_ref[...] = impc

    @pl.when(j != 0)
    def _():
        imp_ref[...] = imp_ref[...] + impc

    @pl.when(j == NPAIR - 1)
    def _():
        vimp = imp_ref[...]
        jj = jax.lax.broadcasted_iota(jnp.int32, (tq, nb), 1)
        m1 = jnp.max(vimp, -1, keepdims=True)
        i1 = jnp.min(jnp.where(vimp == m1, jj, nb), -1, keepdims=True)
        v2 = jnp.where(jj == i1, -jnp.inf, vimp)
        m2 = jnp.max(v2, -1, keepdims=True)
        i2 = jnp.min(jnp.where(v2 == m2, jj, nb), -1, keepdims=True)
        sel_ref[0] = jnp.concatenate([i1, i2], axis=-1)


def _k3(qkv3, ck, cv, mem_kp, mem_vp, qt, tq=512):
    b = qkv3.shape[0]
    nb = (qt + 1) * tq // CBS

    def body(*refs):
        _k3_body(*refs, tq=tq, nb=nb, qt=qt)

    return pl.pallas_call(
        body,
        grid=(b, NPAIR),
        in_specs=[
            pl.BlockSpec((1, tq, 2 * DH), lambda bi, j: (bi, qt, j)),
            pl.BlockSpec((1, nb, 2 * DH), lambda bi, j: (bi, 0, j)),
            pl.BlockSpec((1, nb, 2 * DH), lambda bi, j: (bi, 0, j)),
            pl.BlockSpec((1, 1, 2 * DH), lambda bi, j: (j, 0, 0)),
            pl.BlockSpec((1, 1, 2 * DH), lambda bi, j: (j, 0, 0)),
        ],
        out_specs=[
            pl.BlockSpec((1, tq, 2 * DH), lambda bi, j: (bi, 0, j)),
            pl.BlockSpec((1, tq, NSEL), lambda bi, j: (bi, 0, 0)),
        ],
        out_shape=[
            jax.ShapeDtypeStruct((b, tq, INNER), jnp.float32),
            jax.ShapeDtypeStruct((b, tq, NSEL), jnp.int32),
        ],
        scratch_shapes=[pltpu.VMEM((tq, nb), jnp.float32)],
    )(qkv3, ck, cv, mem_kp, mem_vp)


def _k5_body(q_ref, k_ref, v_ref, cout_ref, sel_ref, g_ref, out_ref,
             wc_ref, *, tq, nw, qt):
    j = pl.program_id(1)
    qs = qt * tq

    @pl.when(j == 0)
    def _():
        i2 = qs + jax.lax.broadcasted_iota(jnp.int32, (tq, nw), 0)
        j2 = jax.lax.broadcasted_iota(jnp.int32, (tq, nw), 1)
        jblk = j2 // SBS
        sel = sel_ref[0]
        sel0 = sel[:, 0:1]
        sel1 = sel[:, 1:2]
        icol0 = qs + jax.lax.broadcasted_iota(jnp.int32, (tq, 1), 0)
        own = icol0 // SBS
        w = ((jblk == sel0).astype(jnp.float32)
             + (jblk == sel1).astype(jnp.float32)
             + (jblk == own).astype(jnp.float32))
        causal = j2 <= i2
        wc_ref[...] = jnp.where(causal, w, 0.0)

    q2 = q_ref[0]
    k2 = k_ref[0]
    v2 = v_ref[0]
    icol = qs + jax.lax.broadcasted_iota(jnp.int32, (tq, 1), 0)
    kti = k_ref[0, pl.ds(qs, tq), :]
    vti = v_ref[0, pl.ds(qs, tq), :]
    pstart = jnp.maximum(qs - 1, 0)
    kpi = jnp.concatenate([k_ref[0, pl.ds(pstart, 1), :], kti[:-1]], axis=0)
    vpi = jnp.concatenate([v_ref[0, pl.ds(pstart, 1), :], vti[:-1]], axis=0)
    wcm = wc_ref[...]
    outs = []
    for hh in range(2):
        sl = slice(hh * DH, (hh + 1) * DH)
        q = q2[:, sl]
        k = k2[:, sl]
        v = v2[:, sl]
        s = jax.lax.dot_general(q.astype(jnp.bfloat16), k.astype(jnp.bfloat16),
                                (((1,), (1,)), ((), ())),
                                preferred_element_type=jnp.float32) * SCALE
        mx = jnp.max(s, -1, keepdims=True)
        e = wcm * jnp.exp(s - mx)
        den = jnp.sum(e, -1, keepdims=True)
        sout = jnp.dot(e.astype(jnp.bfloat16), v.astype(jnp.bfloat16),
                       preferred_element_type=jnp.float32) / den
        kt = kti[:, sl]
        vt = vti[:, sl]
        kprev = kpi[:, sl]
        vprev = vpi[:, sl]
        s1 = jnp.sum(q * kt, -1, keepdims=True) * SCALE
        s0 = jnp.sum(q * kprev, -1, keepdims=True) * SCALE
        s0 = jnp.where(icol > 0, s0, NEG)
        mw = jnp.maximum(s0, s1)
        e0 = jnp.exp(s0 - mw)
        e1 = jnp.exp(s1 - mw)
        wout = (e0 * vprev + e1 * vt) / (e0 + e1)
        g = g_ref[0, hh]
        outs.append(g[:, 0:1] * cout_ref[0][:, sl] + g[:, 1:2] * sout
                    + g[:, 2:3] * wout)
    out_ref[0] = jnp.concatenate(outs, axis=-1)


def _k5(qkv3, cout, sel, g3p, qt, tq=512):
    b, n, _ = qkv3.shape
    nw = (qt + 1) * tq

    def body(*refs):
        _k5_body(*refs, tq=tq, nw=nw, qt=qt)

    return pl.pallas_call(
        body,
        grid=(b, NPAIR),
        in_specs=[
            pl.BlockSpec((1, tq, 2 * DH), lambda bi, j: (bi, qt, j)),
            pl.BlockSpec((1, nw, 2 * DH), lambda bi, j: (bi, 0, NPAIR + j)),
            pl.BlockSpec((1, nw, 2 * DH), lambda bi, j: (bi, 0, 2 * NPAIR + j)),
            pl.BlockSpec((1, tq, 2 * DH), lambda bi, j: (bi, 0, j)),
            pl.BlockSpec((1, tq, NSEL), lambda bi, j: (bi, 0, 0)),
            pl.BlockSpec((1, 2, tq, 3), lambda bi, j: (bi, j, qt, 0)),
        ],
        out_specs=pl.BlockSpec((1, tq, 2 * DH), lambda bi, j: (bi, 0, j)),
        out_shape=jax.ShapeDtypeStruct((b, tq, INNER), jnp.float32),
        scratch_shapes=[pltpu.VMEM((tq, nw), jnp.float32)],
    )(qkv3, qkv3, qkv3, cout, sel, g3p)


def _k6_body(a_ref, w_ref, r_ref, o_ref):
    o_ref[...] = (jnp.dot(a_ref[...].astype(jnp.bfloat16), w_ref[...],
                          preferred_element_type=jnp.float32) + r_ref[...])


def _k6(a, w, res, tr=512):
    r, d_in = a.shape
    d_out = w.shape[1]
    return pl.pallas_call(
        _k6_body,
        grid=(r // tr,),
        in_specs=[
            pl.BlockSpec((tr, d_in), lambda i: (i, 0)),
            pl.BlockSpec((d_in, d_out), lambda i: (0, 0)),
            pl.BlockSpec((tr, d_out), lambda i: (i, 0)),
        ],
        out_specs=pl.BlockSpec((tr, d_out), lambda i: (i, 0)),
        out_shape=jax.ShapeDtypeStruct((r, d_out), jnp.float32),
    )(a, w, res)


def _k7_body(x_ref, g2_ref, b2_ref, w_ref, b_ref, o_ref):
    x = x_ref[...]
    m = jnp.mean(x, -1, keepdims=True)
    v = jnp.mean((x - m) ** 2, -1, keepdims=True)
    xn = (x - m) / jnp.sqrt(v + 1e-5) * g2_ref[...] + b2_ref[...]
    z = jnp.dot(xn.astype(jnp.bfloat16), w_ref[...],
                preferred_element_type=jnp.float32) + b_ref[...]
    o_ref[...] = jnp.where(z >= 0, z, 0.01 * z).astype(jnp.bfloat16)


def _k7(x2, ln_g, ln_b, w1, b1, tr=512):
    r, d_in = x2.shape
    d_out = w1.shape[1]
    return pl.pallas_call(
        _k7_body,
        grid=(r // tr,),
        in_specs=[
            pl.BlockSpec((tr, d_in), lambda i: (i, 0)),
            pl.BlockSpec((1, d_in), lambda i: (0, 0)),
            pl.BlockSpec((1, d_in), lambda i: (0, 0)),
            pl.BlockSpec((d_in, d_out), lambda i: (0, 0)),
            pl.BlockSpec((1, d_out), lambda i: (0, 0)),
        ],
        out_specs=pl.BlockSpec((tr, d_out), lambda i: (i, 0)),
        out_shape=jax.ShapeDtypeStruct((r, d_out), jnp.bfloat16),
    )(x2, ln_g[None], ln_b[None], w1, b1[None])


def _k8_body(a_ref, w_ref, b_ref, r_ref, o_ref):
    o_ref[...] = (jnp.dot(a_ref[...], w_ref[...],
                          preferred_element_type=jnp.float32)
                  + b_ref[...] + r_ref[...])


def _k8(a, w, bias, res, tr=512):
    r, d_in = a.shape
    d_out = w.shape[1]
    return pl.pallas_call(
        _k8_body,
        grid=(r // tr,),
        in_specs=[
            pl.BlockSpec((tr, d_in), lambda i: (i, 0)),
            pl.BlockSpec((d_in, d_out), lambda i: (0, 0)),
            pl.BlockSpec((1, d_out), lambda i: (0, 0)),
            pl.BlockSpec((tr, d_out), lambda i: (i, 0)),
        ],
        out_specs=pl.BlockSpec((tr, d_out), lambda i: (i, 0)),
        out_shape=jax.ShapeDtypeStruct((r, d_out), jnp.float32),
    )(a, w, bias[None], res)


def _layer(x, p):
    b, n, _ = x.shape
    r = b * n
    nb = n // CBS
    x2 = x.reshape(r, DIM)
    qkv, g36 = _k1(x2, p['ln1_g'], p['ln1_b'], p['Wqkv'], p['Wg'], p['bg'])
    qkv3 = qkv.reshape(b, n, 3 * INNER)
    ck, cv = _k2(qkv3, p['Wkc'], p['Wvc'], p['bkc'], p['bvc'],
                 p['k_pos'], p['v_pos'])
    g3p = g36.reshape(b, n, 3, HEADS).transpose(0, 3, 1, 2)
    mem_kp = p['mem_k'].reshape(NPAIR, 1, 2 * DH)
    mem_vp = p['mem_v'].reshape(NPAIR, 1, 2 * DH)
    tq = 512
    combs = []
    for qt in range(n // tq):
        cout_t, sel_t = _k3(qkv3, ck, cv, mem_kp, mem_vp, qt, tq=tq)
        combs.append(_k5(qkv3, cout_t, sel_t, g3p, qt, tq=tq))
    comb2 = jnp.concatenate(combs, axis=1).reshape(r, INNER)
    y = _k6(comb2, p['Wo'].astype(jnp.bfloat16), x2)
    h1 = _k7(y, p['ln2_g'], p['ln2_b'], p['W1'].astype(jnp.bfloat16), p['b1'])
    out = _k8(h1, p['W2'].astype(jnp.bfloat16), p['b2'], y)
    return out.reshape(b, n, DIM)


def kernel(x, params):
    for p in params:
        x = _layer(x, p)
    return x


# speedup vs baseline: 6.2166x; 1.0113x over previous
import jax
import jax.numpy as jnp
from jax.experimental import pallas as pl
from jax.experimental.pallas import tpu as pltpu

DIM = 768
HEADS = 12
NPAIR = HEADS // 2
DH = 64
INNER = HEADS * DH
MLP_D = 1536
W_WIN = 2
CBS = 4
SBS = 4
NSEL = 2
SCALE = DH ** -0.5
NEG = -1e9


def _k1_body(x_ref, g1_ref, b1_ref, wqkv_ref, wg_ref, bg_ref, qkv_ref, g_ref):
    x = x_ref[...]
    m = jnp.mean(x, -1, keepdims=True)
    v = jnp.mean((x - m) ** 2, -1, keepdims=True)
    xn = (x - m) / jnp.sqrt(v + 1e-5) * g1_ref[...] + b1_ref[...]
    qkv_ref[...] = jnp.dot(xn, wqkv_ref[...], preferred_element_type=jnp.float32)
    g_ref[...] = jax.nn.sigmoid(
        jnp.dot(xn, wg_ref[...], preferred_element_type=jnp.float32) + bg_ref[...])


def _k1(x2, ln_g, ln_b, wqkv, wg, bg, tr=512):
    r = x2.shape[0]
    return pl.pallas_call(
        _k1_body,
        grid=(r // tr,),
        in_specs=[
            pl.BlockSpec((tr, DIM), lambda i: (i, 0)),
            pl.BlockSpec((1, DIM), lambda i: (0, 0)),
            pl.BlockSpec((1, DIM), lambda i: (0, 0)),
            pl.BlockSpec((DIM, 3 * INNER), lambda i: (0, 0)),
            pl.BlockSpec((DIM, 3 * HEADS), lambda i: (0, 0)),
            pl.BlockSpec((1, 3 * HEADS), lambda i: (0, 0)),
        ],
        out_specs=[
            pl.BlockSpec((tr, 3 * INNER), lambda i: (i, 0)),
            pl.BlockSpec((tr, 3 * HEADS), lambda i: (i, 0)),
        ],
        out_shape=[
            jax.ShapeDtypeStruct((r, 3 * INNER), jnp.float32),
            jax.ShapeDtypeStruct((r, 3 * HEADS), jnp.float32),
        ],
    )(x2, ln_g[None], ln_b[None], wqkv, wg, bg[None])


def _k2_body(k0_ref, k1_ref, k2_ref, k3_ref, v0_ref, v1_ref, v2_ref, v3_ref,
             wkc_ref, wvc_ref, bk_ref, bv_ref, kp_ref, vp_ref,
             ck_ref, cv_ref):
    wkc = wkc_ref[...]
    wvc = wvc_ref[...]
    bk = jnp.dot(kp_ref[...], wkc, preferred_element_type=jnp.float32) + bk_ref[...]
    bv = jnp.dot(vp_ref[...], wvc, preferred_element_type=jnp.float32) + bv_ref[...]
    krs = [k0_ref[0], k1_ref[0], k2_ref[0], k3_ref[0]]
    vrs = [v0_ref[0], v1_ref[0], v2_ref[0], v3_ref[0]]
    cks = []
    cvs = []
    for hh in range(2):
        sl = slice(hh * DH, (hh + 1) * DH)
        ck = bk
        cv = bv
        for jr in range(CBS):
            wsl = slice(jr * DH, (jr + 1) * DH)
            ck = ck + jnp.dot(krs[jr][:, sl], wkc[wsl],
                              preferred_element_type=jnp.float32)
            cv = cv + jnp.dot(vrs[jr][:, sl], wvc[wsl],
                              preferred_element_type=jnp.float32)
        cks.append(ck)
        cvs.append(cv)
    ck_ref[0] = jnp.concatenate(cks, axis=-1)
    cv_ref[0] = jnp.concatenate(cvs, axis=-1)


def _k2(qkv3, wkc, wvc, bkc, bvc, kpos, vpos):
    cd = CBS * DH
    b, n, _ = qkv3.shape
    nb = n // CBS
    q4 = qkv3.reshape(b, nb, CBS * 3 * INNER)
    kspec = [pl.BlockSpec((1, nb, 2 * DH),
                          (lambda bi, j, jr_=jr: (bi, 0, 18 * jr_ + NPAIR + j)))
             for jr in range(CBS)]
    vspec = [pl.BlockSpec((1, nb, 2 * DH),
                          (lambda bi, j, jr_=jr: (bi, 0, 18 * jr_ + 2 * NPAIR + j)))
             for jr in range(CBS)]
    return pl.pallas_call(
        _k2_body,
        grid=(b, NPAIR),
        in_specs=kspec + vspec + [
            pl.BlockSpec((cd, DH), lambda bi, j: (0, 0)),
            pl.BlockSpec((cd, DH), lambda bi, j: (0, 0)),
            pl.BlockSpec((1, DH), lambda bi, j: (0, 0)),
            pl.BlockSpec((1, DH), lambda bi, j: (0, 0)),
            pl.BlockSpec((1, cd), lambda bi, j: (0, 0)),
            pl.BlockSpec((1, cd), lambda bi, j: (0, 0)),
        ],
        out_specs=[
            pl.BlockSpec((1, nb, 2 * DH), lambda bi, j: (bi, 0, j)),
            pl.BlockSpec((1, nb, 2 * DH), lambda bi, j: (bi, 0, j)),
        ],
        out_shape=[
            jax.ShapeDtypeStruct((b, nb, INNER), jnp.float32),
            jax.ShapeDtypeStruct((b, nb, INNER), jnp.float32),
        ],
    )(q4, q4, q4, q4, q4, q4, q4, q4, wkc, wvc, bkc[None], bvc[None],
      kpos.reshape(1, cd), vpos.reshape(1, cd))


def _k3_body(q_ref, ck_ref, cv_ref, mk_ref, mv_ref, cout_ref, sel_ref,
             imp_ref, bias_ref, *, tq, nb, qt):
    j = pl.program_id(1)
    q2 = q_ref[0]
    ck2 = ck_ref[0]
    cv2 = cv_ref[0]
    mk2 = mk_ref[0]
    mv2 = mv_ref[0]

    @pl.when(j == 0)
    def _():
        ivec = qt * tq + jax.lax.broadcasted_iota(jnp.int32, (tq, nb), 0)
        mvec = jax.lax.broadcasted_iota(jnp.int32, (tq, nb), 1)
        bias_ref[...] = jnp.where((CBS * mvec + (CBS - 1)) <= ivec, 0.0, NEG)

    bias = bias_ref[...]
    couts = []
    imps = []
    for hh in range(2):
        q = q2[:, hh * DH:(hh + 1) * DH]
        ck = ck2[:, hh * DH:(hh + 1) * DH]
        cv = cv2[:, hh * DH:(hh + 1) * DH]
        s = jax.lax.dot_general(q, ck, (((1,), (1,)), ((), ())),
                                preferred_element_type=jnp.float32) * SCALE
        s = s + bias
        smem = jnp.sum(q * mk2[:, hh * DH:(hh + 1) * DH], -1,
                       keepdims=True) * SCALE
        mx = jnp.maximum(jnp.max(s, -1, keepdims=True), smem)
        eb = jnp.exp(s - mx)
        em = jnp.exp(smem - mx)
        den = jnp.sum(eb, -1, keepdims=True) + em
        couts.append((jnp.dot(eb, cv, preferred_element_type=jnp.float32)
                      + em * mv2[:, hh * DH:(hh + 1) * DH]) / den)
        imps.append(eb / den)
    cout_ref[0] = jnp.concatenate(couts, axis=-1)
    impc = (imps[0] + imps[1]) * (1.0 / HEADS)

    @pl.when(j == 0)
    def _():
        imp_ref[...] = impc

    @pl.when(j != 0)
    def _():
        imp_ref[...] = imp_ref[...] + impc

    @pl.when(j == NPAIR - 1)
    def _():
        vimp = imp_ref[...]
        jj = jax.lax.broadcasted_iota(jnp.int32, (tq, nb), 1)
        m1 = jnp.max(vimp, -1, keepdims=True)
        i1 = jnp.min(jnp.where(vimp == m1, jj, nb), -1, keepdims=True)
        v2 = jnp.where(jj == i1, -jnp.inf, vimp)
        m2 = jnp.max(v2, -1, keepdims=True)
        i2 = jnp.min(jnp.where(v2 == m2, jj, nb), -1, keepdims=True)
        sel_ref[0] = jnp.concatenate([i1, i2], axis=-1)


def _k3(qkv3, ck, cv, mem_kp, mem_vp, qt, tq=512):
    b = qkv3.shape[0]
    nb = (qt + 1) * tq // CBS

    def body(*refs):
        _k3_body(*refs, tq=tq, nb=nb, qt=qt)

    return pl.pallas_call(
        body,
        grid=(b, NPAIR),
        in_specs=[
            pl.BlockSpec((1, tq, 2 * DH), lambda bi, j: (bi, qt, j)),
            pl.BlockSpec((1, nb, 2 * DH), lambda bi, j: (bi, 0, j)),
            pl.BlockSpec((1, nb, 2 * DH), lambda bi, j: (bi, 0, j)),
            pl.BlockSpec((1, 1, 2 * DH), lambda bi, j: (j, 0, 0)),
            pl.BlockSpec((1, 1, 2 * DH), lambda bi, j: (j, 0, 0)),
        ],
        out_specs=[
            pl.BlockSpec((1, tq, 2 * DH), lambda bi, j: (bi, 0, j)),
            pl.BlockSpec((1, tq, NSEL), lambda bi, j: (bi, 0, 0)),
        ],
        out_shape=[
            jax.ShapeDtypeStruct((b, tq, INNER), jnp.float32),
            jax.ShapeDtypeStruct((b, tq, NSEL), jnp.int32),
        ],
        scratch_shapes=[
            pltpu.VMEM((tq, nb), jnp.float32),
            pltpu.VMEM((tq, nb), jnp.float32),
        ],
    )(qkv3, ck, cv, mem_kp, mem_vp)


def _k5_body(q_ref, k_ref, v_ref, cout_ref, sel_ref, g_ref, out_ref,
             wc_ref, *, tq, nw, qt):
    j = pl.program_id(1)
    qs = qt * tq

    @pl.when(j == 0)
    def _():
        i2 = qs + jax.lax.broadcasted_iota(jnp.int32, (tq, nw), 0)
        j2 = jax.lax.broadcasted_iota(jnp.int32, (tq, nw), 1)
        jblk = j2 // SBS
        sel = sel_ref[0]
        sel0 = sel[:, 0:1]
        sel1 = sel[:, 1:2]
        icol0 = qs + jax.lax.broadcasted_iota(jnp.int32, (tq, 1), 0)
        own = icol0 // SBS
        w = ((jblk == sel0).astype(jnp.float32)
             + (jblk == sel1).astype(jnp.float32)
             + (jblk == own).astype(jnp.float32))
        causal = j2 <= i2
        wc_ref[...] = jnp.where(causal, w, 0.0)

    q2 = q_ref[0]
    k2 = k_ref[0]
    v2 = v_ref[0]
    icol = qs + jax.lax.broadcasted_iota(jnp.int32, (tq, 1), 0)
    kti = k_ref[0, pl.ds(qs, tq), :]
    vti = v_ref[0, pl.ds(qs, tq), :]
    pstart = jnp.maximum(qs - 1, 0)
    kpi = jnp.concatenate([k_ref[0, pl.ds(pstart, 1), :], kti[:-1]], axis=0)
    vpi = jnp.concatenate([v_ref[0, pl.ds(pstart, 1), :], vti[:-1]], axis=0)
    wcm = wc_ref[...]
    outs = []
    for hh in range(2):
        sl = slice(hh * DH, (hh + 1) * DH)
        q = q2[:, sl]
        k = k2[:, sl]
        v = v2[:, sl]
        s = jax.lax.dot_general(q.astype(jnp.bfloat16), k.astype(jnp.bfloat16),
                                (((1,), (1,)), ((), ())),
                                preferred_element_type=jnp.float32) * SCALE
        mx = jnp.max(s, -1, keepdims=True)
        e = wcm * jnp.exp(s - mx)
        den = jnp.sum(e, -1, keepdims=True)
        sout = jnp.dot(e.astype(jnp.bfloat16), v.astype(jnp.bfloat16),
                       preferred_element_type=jnp.float32) / den
        kt = kti[:, sl]
        vt = vti[:, sl]
        kprev = kpi[:, sl]
        vprev = vpi[:, sl]
        s1 = jnp.sum(q * kt, -1, keepdims=True) * SCALE
        s0 = jnp.sum(q * kprev, -1, keepdims=True) * SCALE
        s0 = jnp.where(icol > 0, s0, NEG)
        mw = jnp.maximum(s0, s1)
        e0 = jnp.exp(s0 - mw)
        e1 = jnp.exp(s1 - mw)
        wout = (e0 * vprev + e1 * vt) / (e0 + e1)
        g = g_ref[0, hh]
        outs.append(g[:, 0:1] * cout_ref[0][:, sl] + g[:, 1:2] * sout
                    + g[:, 2:3] * wout)
    out_ref[0] = jnp.concatenate(outs, axis=-1)


def _k5(qkv3, cout, sel, g3p, qt, tq=512):
    b, n, _ = qkv3.shape
    nw = (qt + 1) * tq

    def body(*refs):
        _k5_body(*refs, tq=tq, nw=nw, qt=qt)

    return pl.pallas_call(
        body,
        grid=(b, NPAIR),
        in_specs=[
            pl.BlockSpec((1, tq, 2 * DH), lambda bi, j: (bi, qt, j)),
            pl.BlockSpec((1, nw, 2 * DH), lambda bi, j: (bi, 0, NPAIR + j)),
            pl.BlockSpec((1, nw, 2 * DH), lambda bi, j: (bi, 0, 2 * NPAIR + j)),
            pl.BlockSpec((1, tq, 2 * DH), lambda bi, j: (bi, 0, j)),
            pl.BlockSpec((1, tq, NSEL), lambda bi, j: (bi, 0, 0)),
            pl.BlockSpec((1, 2, tq, 3), lambda bi, j: (bi, j, qt, 0)),
        ],
        out_specs=pl.BlockSpec((1, tq, 2 * DH), lambda bi, j: (bi, 0, j)),
        out_shape=jax.ShapeDtypeStruct((b, tq, INNER), jnp.float32),
        scratch_shapes=[pltpu.VMEM((tq, nw), jnp.float32)],
    )(qkv3, qkv3, qkv3, cout, sel, g3p)


def _k67_body(a_ref, wo_ref, r_ref, g2_ref, b2_ref, w1_ref, b1_ref,
              y_ref, h_ref):
    y = (jnp.dot(a_ref[...].astype(jnp.bfloat16), wo_ref[...],
                 preferred_element_type=jnp.float32) + r_ref[...])
    y_ref[...] = y
    m = jnp.mean(y, -1, keepdims=True)
    v = jnp.mean((y - m) ** 2, -1, keepdims=True)
    xn = (y - m) / jnp.sqrt(v + 1e-5) * g2_ref[...] + b2_ref[...]
    z = jnp.dot(xn.astype(jnp.bfloat16), w1_ref[...],
                preferred_element_type=jnp.float32) + b1_ref[...]
    h_ref[...] = jnp.where(z >= 0, z, 0.01 * z).astype(jnp.bfloat16)


def _k67(a, wo, res, ln_g, ln_b, w1, b1, tr=512):
    r, d_in = a.shape

    return pl.pallas_call(
        _k67_body,
        grid=(r // tr,),
        in_specs=[
            pl.BlockSpec((tr, d_in), lambda i: (i, 0)),
            pl.BlockSpec((d_in, DIM), lambda i: (0, 0)),
            pl.BlockSpec((tr, DIM), lambda i: (i, 0)),
            pl.BlockSpec((1, DIM), lambda i: (0, 0)),
            pl.BlockSpec((1, DIM), lambda i: (0, 0)),
            pl.BlockSpec((DIM, MLP_D), lambda i: (0, 0)),
            pl.BlockSpec((1, MLP_D), lambda i: (0, 0)),
        ],
        out_specs=[
            pl.BlockSpec((tr, DIM), lambda i: (i, 0)),
            pl.BlockSpec((tr, MLP_D), lambda i: (i, 0)),
        ],
        out_shape=[
            jax.ShapeDtypeStruct((r, DIM), jnp.float32),
            jax.ShapeDtypeStruct((r, MLP_D), jnp.bfloat16),
        ],
    )(a, wo, res, ln_g[None], ln_b[None], w1, b1[None])


def _k8_body(a_ref, w_ref, b_ref, r_ref, o_ref):
    o_ref[...] = (jnp.dot(a_ref[...], w_ref[...],
                          preferred_element_type=jnp.float32)
                  + b_ref[...] + r_ref[...])


def _k8(a, w, bias, res, tr=512):
    r, d_in = a.shape
    d_out = w.shape[1]
    return pl.pallas_call(
        _k8_body,
        grid=(r // tr,),
        in_specs=[
            pl.BlockSpec((tr, d_in), lambda i: (i, 0)),
            pl.BlockSpec((d_in, d_out), lambda i: (0, 0)),
            pl.BlockSpec((1, d_out), lambda i: (0, 0)),
            pl.BlockSpec((tr, d_out), lambda i: (i, 0)),
        ],
        out_specs=pl.BlockSpec((tr, d_out), lambda i: (i, 0)),
        out_shape=jax.ShapeDtypeStruct((r, d_out), jnp.float32),
    )(a, w, bias[None], res)


def _k81_body(a_ref, w2_ref, b2_ref, r_ref, g1_ref, b1_ref,
              wqkv_ref, wg_ref, bg_ref, x_ref, qkv_ref, g_ref):
    x = (jnp.dot(a_ref[...], w2_ref[...], preferred_element_type=jnp.float32)
         + b2_ref[...] + r_ref[...])
    x_ref[...] = x
    m = jnp.mean(x, -1, keepdims=True)
    v = jnp.mean((x - m) ** 2, -1, keepdims=True)
    xn = (x - m) / jnp.sqrt(v + 1e-5) * g1_ref[...] + b1_ref[...]
    qkv_ref[...] = jnp.dot(xn, wqkv_ref[...], preferred_element_type=jnp.float32)
    g_ref[...] = jax.nn.sigmoid(
        jnp.dot(xn, wg_ref[...], preferred_element_type=jnp.float32) + bg_ref[...])


def _k81(a, w2, b2, res, ln_g, ln_b, wqkv, wg, bg, tr=512):
    r, d_in = a.shape
    return pl.pallas_call(
        _k81_body,
        grid=(r // tr,),
        in_specs=[
            pl.BlockSpec((tr, d_in), lambda i: (i, 0)),
            pl.BlockSpec((d_in, DIM), lambda i: (0, 0)),
            pl.BlockSpec((1, DIM), lambda i: (0, 0)),
            pl.BlockSpec((tr, DIM), lambda i: (i, 0)),
            pl.BlockSpec((1, DIM), lambda i: (0, 0)),
            pl.BlockSpec((1, DIM), lambda i: (0, 0)),
            pl.BlockSpec((DIM, 3 * INNER), lambda i: (0, 0)),
            pl.BlockSpec((DIM, 3 * HEADS), lambda i: (0, 0)),
            pl.BlockSpec((1, 3 * HEADS), lambda i: (0, 0)),
        ],
        out_specs=[
            pl.BlockSpec((tr, DIM), lambda i: (i, 0)),
            pl.BlockSpec((tr, 3 * INNER), lambda i: (i, 0)),
            pl.BlockSpec((tr, 3 * HEADS), lambda i: (i, 0)),
        ],
        out_shape=[
            jax.ShapeDtypeStruct((r, DIM), jnp.float32),
            jax.ShapeDtypeStruct((r, 3 * INNER), jnp.float32),
            jax.ShapeDtypeStruct((r, 3 * HEADS), jnp.float32),
        ],
    )(a, w2, b2[None], res, ln_g[None], ln_b[None], wqkv, wg, bg[None])


def _attn(qkv, g36, p, b, n):
    qkv3 = qkv.reshape(b, n, 3 * INNER)
    ck, cv = _k2(qkv3, p['Wkc'], p['Wvc'], p['bkc'], p['bvc'],
                 p['k_pos'], p['v_pos'])
    g3p = g36.reshape(b, n, 3, HEADS).transpose(0, 3, 1, 2)
    mem_kp = p['mem_k'].reshape(NPAIR, 1, 2 * DH)
    mem_vp = p['mem_v'].reshape(NPAIR, 1, 2 * DH)
    tq = 512
    combs = []
    for qt in range(n // tq):
        cout_t, sel_t = _k3(qkv3, ck, cv, mem_kp, mem_vp, qt, tq=tq)
        combs.append(_k5(qkv3, cout_t, sel_t, g3p, qt, tq=tq))
    return jnp.concatenate(combs, axis=1).reshape(b * n, INNER)


def kernel(x, params):
    b, n, _ = x.shape
    p0, p1 = params[0], params[1]
    x2 = x.reshape(b * n, DIM)
    qkv, g36 = _k1(x2, p0['ln1_g'], p0['ln1_b'], p0['Wqkv'], p0['Wg'],
                   p0['bg'])
    comb2 = _attn(qkv, g36, p0, b, n)
    y0, h0 = _k67(comb2, p0['Wo'].astype(jnp.bfloat16), x2,
                  p0['ln2_g'], p0['ln2_b'],
                  p0['W1'].astype(jnp.bfloat16), p0['b1'])
    x1, qkv1, g36_1 = _k81(h0, p0['W2'].astype(jnp.bfloat16), p0['b2'], y0,
                           p1['ln1_g'], p1['ln1_b'], p1['Wqkv'], p1['Wg'],
                           p1['bg'])
    comb2_1 = _attn(qkv1, g36_1, p1, b, n)
    y1, h1 = _k67(comb2_1, p1['Wo'].astype(jnp.bfloat16), x1,
                  p1['ln2_g'], p1['ln2_b'],
                  p1['W1'].astype(jnp.bfloat16), p1['b1'])
    out = _k8(h1, p1['W2'].astype(jnp.bfloat16), p1['b2'], y1)
    return out.reshape(b, n, DIM)


# speedup vs baseline: 6.2352x; 1.0030x over previous
import jax
import jax.numpy as jnp
from jax.experimental import pallas as pl
from jax.experimental.pallas import tpu as pltpu

DIM = 768
HEADS = 12
NPAIR = HEADS // 2
DH = 64
INNER = HEADS * DH
MLP_D = 1536
W_WIN = 2
CBS = 4
SBS = 4
NSEL = 2
SCALE = DH ** -0.5
NEG = -1e9


def _k1_body(x_ref, g1_ref, b1_ref, wqkv_ref, wg_ref, bg_ref, qkv_ref, g_ref):
    x = x_ref[...]
    m = jnp.mean(x, -1, keepdims=True)
    v = jnp.mean((x - m) ** 2, -1, keepdims=True)
    xn = (x - m) / jnp.sqrt(v + 1e-5) * g1_ref[...] + b1_ref[...]
    qkv_ref[...] = jnp.dot(xn, wqkv_ref[...], preferred_element_type=jnp.float32)
    g_ref[...] = jax.nn.sigmoid(
        jnp.dot(xn, wg_ref[...], preferred_element_type=jnp.float32) + bg_ref[...])


def _k1(x2, ln_g, ln_b, wqkv, wg, bg, tr=1024):
    r = x2.shape[0]
    return pl.pallas_call(
        _k1_body,
        grid=(r // tr,),
        in_specs=[
            pl.BlockSpec((tr, DIM), lambda i: (i, 0)),
            pl.BlockSpec((1, DIM), lambda i: (0, 0)),
            pl.BlockSpec((1, DIM), lambda i: (0, 0)),
            pl.BlockSpec((DIM, 3 * INNER), lambda i: (0, 0)),
            pl.BlockSpec((DIM, 3 * HEADS), lambda i: (0, 0)),
            pl.BlockSpec((1, 3 * HEADS), lambda i: (0, 0)),
        ],
        out_specs=[
            pl.BlockSpec((tr, 3 * INNER), lambda i: (i, 0)),
            pl.BlockSpec((tr, 3 * HEADS), lambda i: (i, 0)),
        ],
        out_shape=[
            jax.ShapeDtypeStruct((r, 3 * INNER), jnp.float32),
            jax.ShapeDtypeStruct((r, 3 * HEADS), jnp.float32),
        ],
    )(x2, ln_g[None], ln_b[None], wqkv, wg, bg[None])


def _k2_body(k0_ref, k1_ref, k2_ref, k3_ref, v0_ref, v1_ref, v2_ref, v3_ref,
             wkc_ref, wvc_ref, bk_ref, bv_ref, kp_ref, vp_ref,
             ck_ref, cv_ref):
    wkc = wkc_ref[...]
    wvc = wvc_ref[...]
    bk = jnp.dot(kp_ref[...], wkc, preferred_element_type=jnp.float32) + bk_ref[...]
    bv = jnp.dot(vp_ref[...], wvc, preferred_element_type=jnp.float32) + bv_ref[...]
    krs = [k0_ref[0], k1_ref[0], k2_ref[0], k3_ref[0]]
    vrs = [v0_ref[0], v1_ref[0], v2_ref[0], v3_ref[0]]
    cks = []
    cvs = []
    for hh in range(2):
        sl = slice(hh * DH, (hh + 1) * DH)
        ck = bk
        cv = bv
        for jr in range(CBS):
            wsl = slice(jr * DH, (jr + 1) * DH)
            ck = ck + jnp.dot(krs[jr][:, sl], wkc[wsl],
                              preferred_element_type=jnp.float32)
            cv = cv + jnp.dot(vrs[jr][:, sl], wvc[wsl],
                              preferred_element_type=jnp.float32)
        cks.append(ck)
        cvs.append(cv)
    ck_ref[0] = jnp.concatenate(cks, axis=-1)
    cv_ref[0] = jnp.concatenate(cvs, axis=-1)


def _k2(qkv3, wkc, wvc, bkc, bvc, kpos, vpos):
    cd = CBS * DH
    b, n, _ = qkv3.shape
    nb = n // CBS
    q4 = qkv3.reshape(b, nb, CBS * 3 * INNER)
    kspec = [pl.BlockSpec((1, nb, 2 * DH),
                          (lambda bi, j, jr_=jr: (bi, 0, 18 * jr_ + NPAIR + j)))
             for jr in range(CBS)]
    vspec = [pl.BlockSpec((1, nb, 2 * DH),
                          (lambda bi, j, jr_=jr: (bi, 0, 18 * jr_ + 2 * NPAIR + j)))
             for jr in range(CBS)]
    return pl.pallas_call(
        _k2_body,
        grid=(b, NPAIR),
        in_specs=kspec + vspec + [
            pl.BlockSpec((cd, DH), lambda bi, j: (0, 0)),
            pl.BlockSpec((cd, DH), lambda bi, j: (0, 0)),
            pl.BlockSpec((1, DH), lambda bi, j: (0, 0)),
            pl.BlockSpec((1, DH), lambda bi, j: (0, 0)),
            pl.BlockSpec((1, cd), lambda bi, j: (0, 0)),
            pl.BlockSpec((1, cd), lambda bi, j: (0, 0)),
        ],
        out_specs=[
            pl.BlockSpec((1, nb, 2 * DH), lambda bi, j: (bi, 0, j)),
            pl.BlockSpec((1, nb, 2 * DH), lambda bi, j: (bi, 0, j)),
        ],
        out_shape=[
            jax.ShapeDtypeStruct((b, nb, INNER), jnp.float32),
            jax.ShapeDtypeStruct((b, nb, INNER), jnp.float32),
        ],
    )(q4, q4, q4, q4, q4, q4, q4, q4, wkc, wvc, bkc[None], bvc[None],
      kpos.reshape(1, cd), vpos.reshape(1, cd))


def _k3_body(q_ref, ck_ref, cv_ref, mk_ref, mv_ref, cout_ref, sel_ref,
             imp_ref, bias_ref, *, tq, nb, qt):
    j = pl.program_id(1)
    q2 = q_ref[0]
    ck2 = ck_ref[0]
    cv2 = cv_ref[0]
    mk2 = mk_ref[0]
    mv2 = mv_ref[0]

    @pl.when(j == 0)
    def _():
        ivec = qt * tq + jax.lax.broadcasted_iota(jnp.int32, (tq, nb), 0)
        mvec = jax.lax.broadcasted_iota(jnp.int32, (tq, nb), 1)
        bias_ref[...] = jnp.where((CBS * mvec + (CBS - 1)) <= ivec, 0.0, NEG)

    bias = bias_ref[...]
    couts = []
    imps = []
    for hh in range(2):
        q = q2[:, hh * DH:(hh + 1) * DH]
        ck = ck2[:, hh * DH:(hh + 1) * DH]
        cv = cv2[:, hh * DH:(hh + 1) * DH]
        s = jax.lax.dot_general(q, ck, (((1,), (1,)), ((), ())),
                                preferred_element_type=jnp.float32) * SCALE
        s = s + bias
        smem = jnp.sum(q * mk2[:, hh * DH:(hh + 1) * DH], -1,
                       keepdims=True) * SCALE
        mx = jnp.maximum(jnp.max(s, -1, keepdims=True), smem)
        eb = jnp.exp(s - mx)
        em = jnp.exp(smem - mx)
        den = jnp.sum(eb, -1, keepdims=True) + em
        rc = 1.0 / den
        couts.append((jnp.dot(eb, cv, preferred_element_type=jnp.float32)
                      + em * mv2[:, hh * DH:(hh + 1) * DH]) * rc)
        imps.append(eb * rc)
    cout_ref[0] = jnp.concatenate(couts, axis=-1)
    impc = (imps[0] + imps[1]) * (1.0 / HEADS)

    @pl.when(j == 0)
    def _():
        imp_ref[...] = impc

    @pl.when(j != 0)
    def _():
        imp_ref[...] = imp_ref[...] + impc

    @pl.when(j == NPAIR - 1)
    def _():
        vimp = imp_ref[...]
        jj = jax.lax.broadcasted_iota(jnp.int32, (tq, nb), 1)
        m1 = jnp.max(vimp, -1, keepdims=True)
        i1 = jnp.min(jnp.where(vimp == m1, jj, nb), -1, keepdims=True)
        v2 = jnp.where(jj == i1, -jnp.inf, vimp)
        m2 = jnp.max(v2, -1, keepdims=True)
        i2 = jnp.min(jnp.where(v2 == m2, jj, nb), -1, keepdims=True)
        sel_ref[0] = jnp.concatenate([i1, i2], axis=-1)


def _k3(qkv3, ck, cv, mem_kp, mem_vp, qt, tq=512):
    b = qkv3.shape[0]
    nb = (qt + 1) * tq // CBS

    def body(*refs):
        _k3_body(*refs, tq=tq, nb=nb, qt=qt)

    return pl.pallas_call(
        body,
        grid=(b, NPAIR),
        in_specs=[
            pl.BlockSpec((1, tq, 2 * DH), lambda bi, j: (bi, qt, j)),
            pl.BlockSpec((1, nb, 2 * DH), lambda bi, j: (bi, 0, j)),
            pl.BlockSpec((1, nb, 2 * DH), lambda bi, j: (bi, 0, j)),
            pl.BlockSpec((1, 1, 2 * DH), lambda bi, j: (j, 0, 0)),
            pl.BlockSpec((1, 1, 2 * DH), lambda bi, j: (j, 0, 0)),
        ],
        out_specs=[
            pl.BlockSpec((1, tq, 2 * DH), lambda bi, j: (bi, 0, j)),
            pl.BlockSpec((1, tq, NSEL), lambda bi, j: (bi, 0, 0)),
        ],
        out_shape=[
            jax.ShapeDtypeStruct((b, tq, INNER), jnp.float32),
            jax.ShapeDtypeStruct((b, tq, NSEL), jnp.int32),
        ],
        scratch_shapes=[
            pltpu.VMEM((tq, nb), jnp.float32),
            pltpu.VMEM((tq, nb), jnp.float32),
        ],
    )(qkv3, ck, cv, mem_kp, mem_vp)


def _k5_body(q_ref, k_ref, v_ref, cout_ref, sel_ref, g_ref, out_ref,
             wc_ref, *, tq, nw, qt):
    j = pl.program_id(1)
    qs = qt * tq

    @pl.when(j == 0)
    def _():
        i2 = qs + jax.lax.broadcasted_iota(jnp.int32, (tq, nw), 0)
        j2 = jax.lax.broadcasted_iota(jnp.int32, (tq, nw), 1)
        jblk = j2 // SBS
        sel = sel_ref[0]
        sel0 = sel[:, 0:1]
        sel1 = sel[:, 1:2]
        icol0 = qs + jax.lax.broadcasted_iota(jnp.int32, (tq, 1), 0)
        own = icol0 // SBS
        w = ((jblk == sel0).astype(jnp.float32)
             + (jblk == sel1).astype(jnp.float32)
             + (jblk == own).astype(jnp.float32))
        causal = j2 <= i2
        wc_ref[...] = jnp.where(causal, w, 0.0)

    q2 = q_ref[0]
    k2 = k_ref[0]
    v2 = v_ref[0]
    icol = qs + jax.lax.broadcasted_iota(jnp.int32, (tq, 1), 0)
    kti = k_ref[0, pl.ds(qs, tq), :]
    vti = v_ref[0, pl.ds(qs, tq), :]
    pstart = jnp.maximum(qs - 1, 0)
    kpi = jnp.concatenate([k_ref[0, pl.ds(pstart, 1), :], kti[:-1]], axis=0)
    vpi = jnp.concatenate([v_ref[0, pl.ds(pstart, 1), :], vti[:-1]], axis=0)
    wcm = wc_ref[...]
    outs = []
    for hh in range(2):
        sl = slice(hh * DH, (hh + 1) * DH)
        q = q2[:, sl]
        k = k2[:, sl]
        v = v2[:, sl]
        s = jax.lax.dot_general(q.astype(jnp.bfloat16), k.astype(jnp.bfloat16),
                                (((1,), (1,)), ((), ())),
                                preferred_element_type=jnp.float32) * SCALE
        mx = jnp.max(s, -1, keepdims=True)
        e = wcm * jnp.exp(s - mx)
        den = jnp.sum(e, -1, keepdims=True)
        sout = jnp.dot(e.astype(jnp.bfloat16), v.astype(jnp.bfloat16),
                       preferred_element_type=jnp.float32) * (1.0 / den)
        kt = kti[:, sl]
        vt = vti[:, sl]
        kprev = kpi[:, sl]
        vprev = vpi[:, sl]
        s1 = jnp.sum(q * kt, -1, keepdims=True) * SCALE
        s0 = jnp.sum(q * kprev, -1, keepdims=True) * SCALE
        s0 = jnp.where(icol > 0, s0, NEG)
        mw = jnp.maximum(s0, s1)
        e0 = jnp.exp(s0 - mw)
        e1 = jnp.exp(s1 - mw)
        wout = (e0 * vprev + e1 * vt) * (1.0 / (e0 + e1))
        g = g_ref[0, hh]
        outs.append(g[:, 0:1] * cout_ref[0][:, sl] + g[:, 1:2] * sout
                    + g[:, 2:3] * wout)
    out_ref[0] = jnp.concatenate(outs, axis=-1)


def _k5(qkv3, cout, sel, g3p, qt, tq=512):
    b, n, _ = qkv3.shape
    nw = (qt + 1) * tq

    def body(*refs):
        _k5_body(*refs, tq=tq, nw=nw, qt=qt)

    return pl.pallas_call(
        body,
        grid=(b, NPAIR),
        in_specs=[
            pl.BlockSpec((1, tq, 2 * DH), lambda bi, j: (bi, qt, j)),
            pl.BlockSpec((1, nw, 2 * DH), lambda bi, j: (bi, 0, NPAIR + j)),
            pl.BlockSpec((1, nw, 2 * DH), lambda bi, j: (bi, 0, 2 * NPAIR + j)),
            pl.BlockSpec((1, tq, 2 * DH), lambda bi, j: (bi, 0, j)),
            pl.BlockSpec((1, tq, NSEL), lambda bi, j: (bi, 0, 0)),
            pl.BlockSpec((1, 2, tq, 3), lambda bi, j: (bi, j, qt, 0)),
        ],
        out_specs=pl.BlockSpec((1, tq, 2 * DH), lambda bi, j: (bi, 0, j)),
        out_shape=jax.ShapeDtypeStruct((b, tq, INNER), jnp.float32),
        scratch_shapes=[pltpu.VMEM((tq, nw), jnp.float32)],
    )(qkv3, qkv3, qkv3, cout, sel, g3p)


def _k67_body(a_ref, wo_ref, r_ref, g2_ref, b2_ref, w1_ref, b1_ref,
              y_ref, h_ref):
    y = (jnp.dot(a_ref[...].astype(jnp.bfloat16), wo_ref[...],
                 preferred_element_type=jnp.float32) + r_ref[...])
    y_ref[...] = y
    m = jnp.mean(y, -1, keepdims=True)
    v = jnp.mean((y - m) ** 2, -1, keepdims=True)
    xn = (y - m) / jnp.sqrt(v + 1e-5) * g2_ref[...] + b2_ref[...]
    z = jnp.dot(xn.astype(jnp.bfloat16), w1_ref[...],
                preferred_element_type=jnp.float32) + b1_ref[...]
    h_ref[...] = jnp.where(z >= 0, z, 0.01 * z).astype(jnp.bfloat16)


def _k67(a, wo, res, ln_g, ln_b, w1, b1, tr=1024):
    r, d_in = a.shape

    return pl.pallas_call(
        _k67_body,
        grid=(r // tr,),
        in_specs=[
            pl.BlockSpec((tr, d_in), lambda i: (i, 0)),
            pl.BlockSpec((d_in, DIM), lambda i: (0, 0)),
            pl.BlockSpec((tr, DIM), lambda i: (i, 0)),
            pl.BlockSpec((1, DIM), lambda i: (0, 0)),
            pl.BlockSpec((1, DIM), lambda i: (0, 0)),
            pl.BlockSpec((DIM, MLP_D), lambda i: (0, 0)),
            pl.BlockSpec((1, MLP_D), lambda i: (0, 0)),
        ],
        out_specs=[
            pl.BlockSpec((tr, DIM), lambda i: (i, 0)),
            pl.BlockSpec((tr, MLP_D), lambda i: (i, 0)),
        ],
        out_shape=[
            jax.ShapeDtypeStruct((r, DIM), jnp.float32),
            jax.ShapeDtypeStruct((r, MLP_D), jnp.bfloat16),
        ],
    )(a, wo, res, ln_g[None], ln_b[None], w1, b1[None])


def _k8_body(a_ref, w_ref, b_ref, r_ref, o_ref):
    o_ref[...] = (jnp.dot(a_ref[...], w_ref[...],
                          preferred_element_type=jnp.float32)
                  + b_ref[...] + r_ref[...])


def _k8(a, w, bias, res, tr=1024):
    r, d_in = a.shape
    d_out = w.shape[1]
    return pl.pallas_call(
        _k8_body,
        grid=(r // tr,),
        in_specs=[
            pl.BlockSpec((tr, d_in), lambda i: (i, 0)),
            pl.BlockSpec((d_in, d_out), lambda i: (0, 0)),
            pl.BlockSpec((1, d_out), lambda i: (0, 0)),
            pl.BlockSpec((tr, d_out), lambda i: (i, 0)),
        ],
        out_specs=pl.BlockSpec((tr, d_out), lambda i: (i, 0)),
        out_shape=jax.ShapeDtypeStruct((r, d_out), jnp.float32),
    )(a, w, bias[None], res)


def _k81_body(a_ref, w2_ref, b2_ref, r_ref, g1_ref, b1_ref,
              wqkv_ref, wg_ref, bg_ref, x_ref, qkv_ref, g_ref):
    x = (jnp.dot(a_ref[...], w2_ref[...], preferred_element_type=jnp.float32)
         + b2_ref[...] + r_ref[...])
    x_ref[...] = x
    m = jnp.mean(x, -1, keepdims=True)
    v = jnp.mean((x - m) ** 2, -1, keepdims=True)
    xn = (x - m) / jnp.sqrt(v + 1e-5) * g1_ref[...] + b1_ref[...]
    qkv_ref[...] = jnp.dot(xn, wqkv_ref[...], preferred_element_type=jnp.float32)
    g_ref[...] = jax.nn.sigmoid(
        jnp.dot(xn, wg_ref[...], preferred_element_type=jnp.float32) + bg_ref[...])


def _k81(a, w2, b2, res, ln_g, ln_b, wqkv, wg, bg, tr=1024):
    r, d_in = a.shape
    return pl.pallas_call(
        _k81_body,
        grid=(r // tr,),
        in_specs=[
            pl.BlockSpec((tr, d_in), lambda i: (i, 0)),
            pl.BlockSpec((d_in, DIM), lambda i: (0, 0)),
            pl.BlockSpec((1, DIM), lambda i: (0, 0)),
            pl.BlockSpec((tr, DIM), lambda i: (i, 0)),
            pl.BlockSpec((1, DIM), lambda i: (0, 0)),
            pl.BlockSpec((1, DIM), lambda i: (0, 0)),
            pl.BlockSpec((DIM, 3 * INNER), lambda i: (0, 0)),
            pl.BlockSpec((DIM, 3 * HEADS), lambda i: (0, 0)),
            pl.BlockSpec((1, 3 * HEADS), lambda i: (0, 0)),
        ],
        out_specs=[
            pl.BlockSpec((tr, DIM), lambda i: (i, 0)),
            pl.BlockSpec((tr, 3 * INNER), lambda i: (i, 0)),
            pl.BlockSpec((tr, 3 * HEADS), lambda i: (i, 0)),
        ],
        out_shape=[
            jax.ShapeDtypeStruct((r, DIM), jnp.float32),
            jax.ShapeDtypeStruct((r, 3 * INNER), jnp.float32),
            jax.ShapeDtypeStruct((r, 3 * HEADS), jnp.float32),
        ],
    )(a, w2, b2[None], res, ln_g[None], ln_b[None], wqkv, wg, bg[None])


def _attn(qkv, g36, p, b, n):
    qkv3 = qkv.reshape(b, n, 3 * INNER)
    ck, cv = _k2(qkv3, p['Wkc'], p['Wvc'], p['bkc'], p['bvc'],
                 p['k_pos'], p['v_pos'])
    g3p = g36.reshape(b, n, 3, HEADS).transpose(0, 3, 1, 2)
    mem_kp = p['mem_k'].reshape(NPAIR, 1, 2 * DH)
    mem_vp = p['mem_v'].reshape(NPAIR, 1, 2 * DH)
    tq = 512
    combs = []
    for qt in range(n // tq):
        cout_t, sel_t = _k3(qkv3, ck, cv, mem_kp, mem_vp, qt, tq=tq)
        combs.append(_k5(qkv3, cout_t, sel_t, g3p, qt, tq=tq))
    return jnp.concatenate(combs, axis=1).reshape(b * n, INNER)


def kernel(x, params):
    b, n, _ = x.shape
    p0, p1 = params[0], params[1]
    x2 = x.reshape(b * n, DIM)
    qkv, g36 = _k1(x2, p0['ln1_g'], p0['ln1_b'], p0['Wqkv'], p0['Wg'],
                   p0['bg'])
    comb2 = _attn(qkv, g36, p0, b, n)
    y0, h0 = _k67(comb2, p0['Wo'].astype(jnp.bfloat16), x2,
                  p0['ln2_g'], p0['ln2_b'],
                  p0['W1'].astype(jnp.bfloat16), p0['b1'])
    x1, qkv1, g36_1 = _k81(h0, p0['W2'].astype(jnp.bfloat16), p0['b2'], y0,
                           p1['ln1_g'], p1['ln1_b'], p1['Wqkv'], p1['Wg'],
                           p1['bg'])
    comb2_1 = _attn(qkv1, g36_1, p1, b, n)
    y1, h1 = _k67(comb2_1, p1['Wo'].astype(jnp.bfloat16), x1,
                  p1['ln2_g'], p1['ln2_b'],
                  p1['W1'].astype(jnp.bfloat16), p1['b1'])
    out = _k8(h1, p1['W2'].astype(jnp.bfloat16), p1['b2'], y1)
    return out.reshape(b, n, DIM)


# speedup vs baseline: 6.2513x; 1.0026x over previous
import jax
import jax.numpy as jnp
from jax.experimental import pallas as pl
from jax.experimental.pallas import tpu as pltpu

DIM = 768
HEADS = 12
NPAIR = HEADS // 2
DH = 64
INNER = HEADS * DH
MLP_D = 1536
W_WIN = 2
CBS = 4
SBS = 4
NSEL = 2
SCALE = DH ** -0.5
NEG = -1e9


def _k1_body(x_ref, g1_ref, b1_ref, wqkv_ref, wg_ref, bg_ref, qkv_ref,
             qkvb_ref, g_ref):
    x = x_ref[...]
    m = jnp.mean(x, -1, keepdims=True)
    v = jnp.mean((x - m) ** 2, -1, keepdims=True)
    xn = (x - m) / jnp.sqrt(v + 1e-5) * g1_ref[...] + b1_ref[...]
    qv = jnp.dot(xn, wqkv_ref[...], preferred_element_type=jnp.float32)
    qkv_ref[...] = qv
    qkvb_ref[...] = qv.astype(jnp.bfloat16)
    g_ref[...] = jax.nn.sigmoid(
        jnp.dot(xn, wg_ref[...], preferred_element_type=jnp.float32) + bg_ref[...])


def _k1(x2, ln_g, ln_b, wqkv, wg, bg, tr=512):
    r = x2.shape[0]
    return pl.pallas_call(
        _k1_body,
        grid=(r // tr,),
        in_specs=[
            pl.BlockSpec((tr, DIM), lambda i: (i, 0)),
            pl.BlockSpec((1, DIM), lambda i: (0, 0)),
            pl.BlockSpec((1, DIM), lambda i: (0, 0)),
            pl.BlockSpec((DIM, 3 * INNER), lambda i: (0, 0)),
            pl.BlockSpec((DIM, 3 * HEADS), lambda i: (0, 0)),
            pl.BlockSpec((1, 3 * HEADS), lambda i: (0, 0)),
        ],
        out_specs=[
            pl.BlockSpec((tr, 3 * INNER), lambda i: (i, 0)),
            pl.BlockSpec((tr, 3 * INNER), lambda i: (i, 0)),
            pl.BlockSpec((tr, 3 * HEADS), lambda i: (i, 0)),
        ],
        out_shape=[
            jax.ShapeDtypeStruct((r, 3 * INNER), jnp.float32),
            jax.ShapeDtypeStruct((r, 3 * INNER), jnp.bfloat16),
            jax.ShapeDtypeStruct((r, 3 * HEADS), jnp.float32),
        ],
    )(x2, ln_g[None], ln_b[None], wqkv, wg, bg[None])


def _k2_body(k0_ref, k1_ref, k2_ref, k3_ref, v0_ref, v1_ref, v2_ref, v3_ref,
             wkc_ref, wvc_ref, bk_ref, bv_ref, kp_ref, vp_ref,
             ck_ref, cv_ref):
    wkc = wkc_ref[...]
    wvc = wvc_ref[...]
    bk = jnp.dot(kp_ref[...], wkc, preferred_element_type=jnp.float32) + bk_ref[...]
    bv = jnp.dot(vp_ref[...], wvc, preferred_element_type=jnp.float32) + bv_ref[...]
    krs = [k0_ref[0], k1_ref[0], k2_ref[0], k3_ref[0]]
    vrs = [v0_ref[0], v1_ref[0], v2_ref[0], v3_ref[0]]
    cks = []
    cvs = []
    for hh in range(2):
        sl = slice(hh * DH, (hh + 1) * DH)
        ck = bk
        cv = bv
        for jr in range(CBS):
            wsl = slice(jr * DH, (jr + 1) * DH)
            ck = ck + jnp.dot(krs[jr][:, sl], wkc[wsl],
                              preferred_element_type=jnp.float32)
            cv = cv + jnp.dot(vrs[jr][:, sl], wvc[wsl],
                              preferred_element_type=jnp.float32)
        cks.append(ck)
        cvs.append(cv)
    ck_ref[0] = jnp.concatenate(cks, axis=-1)
    cv_ref[0] = jnp.concatenate(cvs, axis=-1)


def _k2(qkv3, wkc, wvc, bkc, bvc, kpos, vpos):
    cd = CBS * DH
    b, n, _ = qkv3.shape
    nb = n // CBS
    q4 = qkv3.reshape(b, nb, CBS * 3 * INNER)
    kspec = [pl.BlockSpec((1, nb, 2 * DH),
                          (lambda bi, j, jr_=jr: (bi, 0, 18 * jr_ + NPAIR + j)))
             for jr in range(CBS)]
    vspec = [pl.BlockSpec((1, nb, 2 * DH),
                          (lambda bi, j, jr_=jr: (bi, 0, 18 * jr_ + 2 * NPAIR + j)))
             for jr in range(CBS)]
    return pl.pallas_call(
        _k2_body,
        grid=(b, NPAIR),
        in_specs=kspec + vspec + [
            pl.BlockSpec((cd, DH), lambda bi, j: (0, 0)),
            pl.BlockSpec((cd, DH), lambda bi, j: (0, 0)),
            pl.BlockSpec((1, DH), lambda bi, j: (0, 0)),
            pl.BlockSpec((1, DH), lambda bi, j: (0, 0)),
            pl.BlockSpec((1, cd), lambda bi, j: (0, 0)),
            pl.BlockSpec((1, cd), lambda bi, j: (0, 0)),
        ],
        out_specs=[
            pl.BlockSpec((1, nb, 2 * DH), lambda bi, j: (bi, 0, j)),
            pl.BlockSpec((1, nb, 2 * DH), lambda bi, j: (bi, 0, j)),
        ],
        out_shape=[
            jax.ShapeDtypeStruct((b, nb, INNER), jnp.float32),
            jax.ShapeDtypeStruct((b, nb, INNER), jnp.float32),
        ],
    )(q4, q4, q4, q4, q4, q4, q4, q4, wkc, wvc, bkc[None], bvc[None],
      kpos.reshape(1, cd), vpos.reshape(1, cd))


def _k3_body(q_ref, ck_ref, cv_ref, mk_ref, mv_ref, cout_ref, sel_ref,
             imp_ref, bias_ref, *, tq, nb, qt):
    j = pl.program_id(1)
    q2 = q_ref[0]
    ck2 = ck_ref[0]
    cv2 = cv_ref[0]
    mk2 = mk_ref[0]
    mv2 = mv_ref[0]

    @pl.when(j == 0)
    def _():
        ivec = qt * tq + jax.lax.broadcasted_iota(jnp.int32, (tq, nb), 0)
        mvec = jax.lax.broadcasted_iota(jnp.int32, (tq, nb), 1)
        bias_ref[...] = jnp.where((CBS * mvec + (CBS - 1)) <= ivec, 0.0, NEG)

    bias = bias_ref[...]
    couts = []
    imps = []
    for hh in range(2):
        q = q2[:, hh * DH:(hh + 1) * DH]
        ck = ck2[:, hh * DH:(hh + 1) * DH]
        cv = cv2[:, hh * DH:(hh + 1) * DH]
        s = jax.lax.dot_general(q, ck, (((1,), (1,)), ((), ())),
                                preferred_element_type=jnp.float32) * SCALE
        s = s + bias
        smem = jnp.sum(q * mk2[:, hh * DH:(hh + 1) * DH], -1,
                       keepdims=True) * SCALE
        mx = jnp.maximum(jnp.max(s, -1, keepdims=True), smem)
        eb = jnp.exp(s - mx)
        em = jnp.exp(smem - mx)
        den = jnp.sum(eb, -1, keepdims=True) + em
        rc = 1.0 / den
        couts.append((jnp.dot(eb, cv, preferred_element_type=jnp.float32)
                      + em * mv2[:, hh * DH:(hh + 1) * DH]) * rc)
        imps.append(eb * rc)
    cout_ref[0] = jnp.concatenate(couts, axis=-1).astype(jnp.bfloat16)
    impc = (imps[0] + imps[1]) * (1.0 / HEADS)

    @pl.when(j == 0)
    def _():
        imp_ref[...] = impc

    @pl.when(j != 0)
    def _():
        imp_ref[...] = imp_ref[...] + impc

    @pl.when(j == NPAIR - 1)
    def _():
        vimp = imp_ref[...]
        jj = jax.lax.broadcasted_iota(jnp.int32, (tq, nb), 1)
        m1 = jnp.max(vimp, -1, keepdims=True)
        i1 = jnp.min(jnp.where(vimp == m1, jj, nb), -1, keepdims=True)
        v2 = jnp.where(jj == i1, -jnp.inf, vimp)
        m2 = jnp.max(v2, -1, keepdims=True)
        i2 = jnp.min(jnp.where(v2 == m2, jj, nb), -1, keepdims=True)
        sel_ref[0] = jnp.concatenate([i1, i2], axis=-1)


def _k3(qkv3, ck, cv, mem_kp, mem_vp, qt, tq=512):
    b = qkv3.shape[0]
    nb = (qt + 1) * tq // CBS

    def body(*refs):
        _k3_body(*refs, tq=tq, nb=nb, qt=qt)

    return pl.pallas_call(
        body,
        grid=(b, NPAIR),
        in_specs=[
            pl.BlockSpec((1, tq, 2 * DH), lambda bi, j: (bi, qt, j)),
            pl.BlockSpec((1, nb, 2 * DH), lambda bi, j: (bi, 0, j)),
            pl.BlockSpec((1, nb, 2 * DH), lambda bi, j: (bi, 0, j)),
            pl.BlockSpec((1, 1, 2 * DH), lambda bi, j: (j, 0, 0)),
            pl.BlockSpec((1, 1, 2 * DH), lambda bi, j: (j, 0, 0)),
        ],
        out_specs=[
            pl.BlockSpec((1, tq, 2 * DH), lambda bi, j: (bi, 0, j)),
            pl.BlockSpec((1, tq, NSEL), lambda bi, j: (bi, 0, 0)),
        ],
        out_shape=[
            jax.ShapeDtypeStruct((b, tq, INNER), jnp.bfloat16),
            jax.ShapeDtypeStruct((b, tq, NSEL), jnp.int32),
        ],
        scratch_shapes=[
            pltpu.VMEM((tq, nb), jnp.float32),
            pltpu.VMEM((tq, nb), jnp.float32),
        ],
    )(qkv3, ck, cv, mem_kp, mem_vp)


def _k5_body(q_ref, k_ref, v_ref, cout_ref, sel_ref, g_ref, out_ref,
             wc_ref, *, tq, nw, qt):
    j = pl.program_id(1)
    qs = qt * tq

    @pl.when(j == 0)
    def _():
        i2 = qs + jax.lax.broadcasted_iota(jnp.int32, (tq, nw), 0)
        j2 = jax.lax.broadcasted_iota(jnp.int32, (tq, nw), 1)
        jblk = j2 // SBS
        sel = sel_ref[0]
        sel0 = sel[:, 0:1]
        sel1 = sel[:, 1:2]
        icol0 = qs + jax.lax.broadcasted_iota(jnp.int32, (tq, 1), 0)
        own = icol0 // SBS
        w = ((jblk == sel0).astype(jnp.float32)
             + (jblk == sel1).astype(jnp.float32)
             + (jblk == own).astype(jnp.float32))
        causal = j2 <= i2
        wc_ref[...] = jnp.where(causal, w, 0.0)

    q2 = q_ref[0]
    k2 = k_ref[0]
    v2 = v_ref[0]
    icol = qs + jax.lax.broadcasted_iota(jnp.int32, (tq, 1), 0)
    kti = k_ref[0, pl.ds(qs, tq), :].astype(jnp.float32)
    vti = v_ref[0, pl.ds(qs, tq), :].astype(jnp.float32)
    pstart = jnp.maximum(qs - 1, 0)
    kp0 = k_ref[0, pl.ds(pstart, 1), :].astype(jnp.float32)
    vp0 = v_ref[0, pl.ds(pstart, 1), :].astype(jnp.float32)
    kpi = jnp.concatenate([kp0, kti[:-1]], axis=0)
    vpi = jnp.concatenate([vp0, vti[:-1]], axis=0)
    wcm = wc_ref[...]
    outs = []
    for hh in range(2):
        sl = slice(hh * DH, (hh + 1) * DH)
        q = q2[:, sl]
        k = k2[:, sl]
        v = v2[:, sl]
        s = jax.lax.dot_general(q, k, (((1,), (1,)), ((), ())),
                                preferred_element_type=jnp.float32) * SCALE
        mx = jnp.max(s, -1, keepdims=True)
        e = wcm * jnp.exp(s - mx)
        den = jnp.sum(e, -1, keepdims=True)
        sout = jnp.dot(e.astype(jnp.bfloat16), v,
                       preferred_element_type=jnp.float32) * (1.0 / den)
        qf = q.astype(jnp.float32)
        kt = kti[:, sl]
        vt = vti[:, sl]
        kprev = kpi[:, sl]
        vprev = vpi[:, sl]
        s1 = jnp.sum(qf * kt, -1, keepdims=True) * SCALE
        s0 = jnp.sum(qf * kprev, -1, keepdims=True) * SCALE
        s0 = jnp.where(icol > 0, s0, NEG)
        mw = jnp.maximum(s0, s1)
        e0 = jnp.exp(s0 - mw)
        e1 = jnp.exp(s1 - mw)
        wout = (e0 * vprev + e1 * vt) * (1.0 / (e0 + e1))
        g = g_ref[0, hh]
        outs.append(g[:, 0:1] * cout_ref[0][:, sl] + g[:, 1:2] * sout
                    + g[:, 2:3] * wout)
    out_ref[0] = jnp.concatenate(outs, axis=-1)


def _k5(qkv3, cout, sel, g3p, qt, tq=512):
    b, n, _ = qkv3.shape
    nw = (qt + 1) * tq

    def body(*refs):
        _k5_body(*refs, tq=tq, nw=nw, qt=qt)

    return pl.pallas_call(
        body,
        grid=(b, NPAIR),
        in_specs=[
            pl.BlockSpec((1, tq, 2 * DH), lambda bi, j: (bi, qt, j)),
            pl.BlockSpec((1, nw, 2 * DH), lambda bi, j: (bi, 0, NPAIR + j)),
            pl.BlockSpec((1, nw, 2 * DH), lambda bi, j: (bi, 0, 2 * NPAIR + j)),
            pl.BlockSpec((1, tq, 2 * DH), lambda bi, j: (bi, 0, j)),
            pl.BlockSpec((1, tq, NSEL), lambda bi, j: (bi, 0, 0)),
            pl.BlockSpec((1, 2, tq, 3), lambda bi, j: (bi, j, qt, 0)),
        ],
        out_specs=pl.BlockSpec((1, tq, 2 * DH), lambda bi, j: (bi, 0, j)),
        out_shape=jax.ShapeDtypeStruct((b, tq, INNER), jnp.float32),
        scratch_shapes=[pltpu.VMEM((tq, nw), jnp.float32)],
    )(qkv3, qkv3, qkv3, cout, sel, g3p)


def _k67_body(a_ref, wo_ref, r_ref, g2_ref, b2_ref, w1_ref, b1_ref,
              y_ref, h_ref):
    y = (jnp.dot(a_ref[...].astype(jnp.bfloat16), wo_ref[...],
                 preferred_element_type=jnp.float32) + r_ref[...])
    y_ref[...] = y
    m = jnp.mean(y, -1, keepdims=True)
    v = jnp.mean((y - m) ** 2, -1, keepdims=True)
    xn = (y - m) / jnp.sqrt(v + 1e-5) * g2_ref[...] + b2_ref[...]
    z = jnp.dot(xn.astype(jnp.bfloat16), w1_ref[...],
                preferred_element_type=jnp.float32) + b1_ref[...]
    h_ref[...] = jnp.where(z >= 0, z, 0.01 * z).astype(jnp.bfloat16)


def _k67(a, wo, res, ln_g, ln_b, w1, b1, tr=1024):
    r, d_in = a.shape

    return pl.pallas_call(
        _k67_body,
        grid=(r // tr,),
        in_specs=[
            pl.BlockSpec((tr, d_in), lambda i: (i, 0)),
            pl.BlockSpec((d_in, DIM), lambda i: (0, 0)),
            pl.BlockSpec((tr, DIM), lambda i: (i, 0)),
            pl.BlockSpec((1, DIM), lambda i: (0, 0)),
            pl.BlockSpec((1, DIM), lambda i: (0, 0)),
            pl.BlockSpec((DIM, MLP_D), lambda i: (0, 0)),
            pl.BlockSpec((1, MLP_D), lambda i: (0, 0)),
        ],
        out_specs=[
            pl.BlockSpec((tr, DIM), lambda i: (i, 0)),
            pl.BlockSpec((tr, MLP_D), lambda i: (i, 0)),
        ],
        out_shape=[
            jax.ShapeDtypeStruct((r, DIM), jnp.float32),
            jax.ShapeDtypeStruct((r, MLP_D), jnp.bfloat16),
        ],
    )(a, wo, res, ln_g[None], ln_b[None], w1, b1[None])


def _k8_body(a_ref, w_ref, b_ref, r_ref, o_ref):
    o_ref[...] = (jnp.dot(a_ref[...], w_ref[...],
                          preferred_element_type=jnp.float32)
                  + b_ref[...] + r_ref[...])


def _k8(a, w, bias, res, tr=1024):
    r, d_in = a.shape
    d_out = w.shape[1]
    return pl.pallas_call(
        _k8_body,
        grid=(r // tr,),
        in_specs=[
            pl.BlockSpec((tr, d_in), lambda i: (i, 0)),
            pl.BlockSpec((d_in, d_out), lambda i: (0, 0)),
            pl.BlockSpec((1, d_out), lambda i: (0, 0)),
            pl.BlockSpec((tr, d_out), lambda i: (i, 0)),
        ],
        out_specs=pl.BlockSpec((tr, d_out), lambda i: (i, 0)),
        out_shape=jax.ShapeDtypeStruct((r, d_out), jnp.float32),
    )(a, w, bias[None], res)


def _k81_body(a_ref, w2_ref, b2_ref, r_ref, g1_ref, b1_ref,
              wqkv_ref, wg_ref, bg_ref, x_ref, qkv_ref, qkvb_ref, g_ref):
    x = (jnp.dot(a_ref[...], w2_ref[...], preferred_element_type=jnp.float32)
         + b2_ref[...] + r_ref[...])
    x_ref[...] = x
    m = jnp.mean(x, -1, keepdims=True)
    v = jnp.mean((x - m) ** 2, -1, keepdims=True)
    xn = (x - m) / jnp.sqrt(v + 1e-5) * g1_ref[...] + b1_ref[...]
    qv = jnp.dot(xn, wqkv_ref[...], preferred_element_type=jnp.float32)
    qkv_ref[...] = qv
    qkvb_ref[...] = qv.astype(jnp.bfloat16)
    g_ref[...] = jax.nn.sigmoid(
        jnp.dot(xn, wg_ref[...], preferred_element_type=jnp.float32) + bg_ref[...])


def _k81(a, w2, b2, res, ln_g, ln_b, wqkv, wg, bg, tr=512):
    r, d_in = a.shape
    return pl.pallas_call(
        _k81_body,
        grid=(r // tr,),
        in_specs=[
            pl.BlockSpec((tr, d_in), lambda i: (i, 0)),
            pl.BlockSpec((d_in, DIM), lambda i: (0, 0)),
            pl.BlockSpec((1, DIM), lambda i: (0, 0)),
            pl.BlockSpec((tr, DIM), lambda i: (i, 0)),
            pl.BlockSpec((1, DIM), lambda i: (0, 0)),
            pl.BlockSpec((1, DIM), lambda i: (0, 0)),
            pl.BlockSpec((DIM, 3 * INNER), lambda i: (0, 0)),
            pl.BlockSpec((DIM, 3 * HEADS), lambda i: (0, 0)),
            pl.BlockSpec((1, 3 * HEADS), lambda i: (0, 0)),
        ],
        out_specs=[
            pl.BlockSpec((tr, DIM), lambda i: (i, 0)),
            pl.BlockSpec((tr, 3 * INNER), lambda i: (i, 0)),
            pl.BlockSpec((tr, 3 * INNER), lambda i: (i, 0)),
            pl.BlockSpec((tr, 3 * HEADS), lambda i: (i, 0)),
        ],
        out_shape=[
            jax.ShapeDtypeStruct((r, DIM), jnp.float32),
            jax.ShapeDtypeStruct((r, 3 * INNER), jnp.float32),
            jax.ShapeDtypeStruct((r, 3 * INNER), jnp.bfloat16),
            jax.ShapeDtypeStruct((r, 3 * HEADS), jnp.float32),
        ],
    )(a, w2, b2[None], res, ln_g[None], ln_b[None], wqkv, wg, bg[None])


def _attn(qkv, qkvb, g36, p, b, n):
    qkv3 = qkv.reshape(b, n, 3 * INNER)
    qkvb3 = qkvb.reshape(b, n, 3 * INNER)
    ck, cv = _k2(qkv3, p['Wkc'], p['Wvc'], p['bkc'], p['bvc'],
                 p['k_pos'], p['v_pos'])
    g3p = g36.reshape(b, n, 3, HEADS).transpose(0, 3, 1, 2)
    mem_kp = p['mem_k'].reshape(NPAIR, 1, 2 * DH)
    mem_vp = p['mem_v'].reshape(NPAIR, 1, 2 * DH)
    tq = 512
    combs = []
    for qt in range(n // tq):
        cout_t, sel_t = _k3(qkv3, ck, cv, mem_kp, mem_vp, qt, tq=tq)
        combs.append(_k5(qkvb3, cout_t, sel_t, g3p, qt, tq=tq))
    return jnp.concatenate(combs, axis=1).reshape(b * n, INNER)


def kernel(x, params):
    b, n, _ = x.shape
    p0, p1 = params[0], params[1]
    x2 = x.reshape(b * n, DIM)
    qkv, qkvb, g36 = _k1(x2, p0['ln1_g'], p0['ln1_b'], p0['Wqkv'], p0['Wg'],
                         p0['bg'])
    comb2 = _attn(qkv, qkvb, g36, p0, b, n)
    y0, h0 = _k67(comb2, p0['Wo'].astype(jnp.bfloat16), x2,
                  p0['ln2_g'], p0['ln2_b'],
                  p0['W1'].astype(jnp.bfloat16), p0['b1'])
    x1, qkv1, qkvb1, g36_1 = _k81(h0, p0['W2'].astype(jnp.bfloat16), p0['b2'],
                                  y0, p1['ln1_g'], p1['ln1_b'], p1['Wqkv'],
                                  p1['Wg'], p1['bg'])
    comb2_1 = _attn(qkv1, qkvb1, g36_1, p1, b, n)
    y1, h1 = _k67(comb2_1, p1['Wo'].astype(jnp.bfloat16), x1,
                  p1['ln2_g'], p1['ln2_b'],
                  p1['W1'].astype(jnp.bfloat16), p1['b1'])
    out = _k8(h1, p1['W2'].astype(jnp.bfloat16), p1['b2'], y1)
    return out.reshape(b, n, DIM)


# speedup vs baseline: 6.4694x; 1.0349x over previous
import jax
import jax.numpy as jnp
from jax.experimental import pallas as pl
from jax.experimental.pallas import tpu as pltpu

DIM = 768
HEADS = 12
NPAIR = HEADS // 2
DH = 64
INNER = HEADS * DH
MLP_D = 1536
W_WIN = 2
CBS = 4
SBS = 4
NSEL = 2
SCALE = DH ** -0.5
NEG = -1e9


def _k1_body(x_ref, g1_ref, b1_ref, wqkv_ref, wg_ref, bg_ref, qkv_ref,
             qkvb_ref, g_ref):
    x = x_ref[...]
    m = jnp.mean(x, -1, keepdims=True)
    v = jnp.mean((x - m) ** 2, -1, keepdims=True)
    xn = (x - m) / jnp.sqrt(v + 1e-5) * g1_ref[...] + b1_ref[...]
    qv = jnp.dot(xn, wqkv_ref[...], preferred_element_type=jnp.float32)
    qkv_ref[...] = qv
    qkvb_ref[...] = qv.astype(jnp.bfloat16)
    g_ref[...] = jax.nn.sigmoid(
        jnp.dot(xn, wg_ref[...], preferred_element_type=jnp.float32) + bg_ref[...])


def _k1(x2, ln_g, ln_b, wqkv, wg, bg, tr=512):
    r = x2.shape[0]
    return pl.pallas_call(
        _k1_body,
        grid=(r // tr,),
        in_specs=[
            pl.BlockSpec((tr, DIM), lambda i: (i, 0)),
            pl.BlockSpec((1, DIM), lambda i: (0, 0)),
            pl.BlockSpec((1, DIM), lambda i: (0, 0)),
            pl.BlockSpec((DIM, 3 * INNER), lambda i: (0, 0)),
            pl.BlockSpec((DIM, 3 * HEADS), lambda i: (0, 0)),
            pl.BlockSpec((1, 3 * HEADS), lambda i: (0, 0)),
        ],
        out_specs=[
            pl.BlockSpec((tr, 3 * INNER), lambda i: (i, 0)),
            pl.BlockSpec((tr, 3 * INNER), lambda i: (i, 0)),
            pl.BlockSpec((tr, 3 * HEADS), lambda i: (i, 0)),
        ],
        out_shape=[
            jax.ShapeDtypeStruct((r, 3 * INNER), jnp.float32),
            jax.ShapeDtypeStruct((r, 3 * INNER), jnp.bfloat16),
            jax.ShapeDtypeStruct((r, 3 * HEADS), jnp.float32),
        ],
    )(x2, ln_g[None], ln_b[None], wqkv, wg, bg[None])


def _k2_body(k0_ref, k1_ref, k2_ref, k3_ref, v0_ref, v1_ref, v2_ref, v3_ref,
             wkc_ref, wvc_ref, bk_ref, bv_ref, kp_ref, vp_ref,
             ck_ref, cv_ref):
    wkc = wkc_ref[...]
    wvc = wvc_ref[...]
    bk = jnp.dot(kp_ref[...], wkc, preferred_element_type=jnp.float32) + bk_ref[...]
    bv = jnp.dot(vp_ref[...], wvc, preferred_element_type=jnp.float32) + bv_ref[...]
    krs = [k0_ref[0], k1_ref[0], k2_ref[0], k3_ref[0]]
    vrs = [v0_ref[0], v1_ref[0], v2_ref[0], v3_ref[0]]
    cks = []
    cvs = []
    for hh in range(2):
        sl = slice(hh * DH, (hh + 1) * DH)
        ck = bk
        cv = bv
        for jr in range(CBS):
            wsl = slice(jr * DH, (jr + 1) * DH)
            ck = ck + jnp.dot(krs[jr][:, sl], wkc[wsl],
                              preferred_element_type=jnp.float32)
            cv = cv + jnp.dot(vrs[jr][:, sl], wvc[wsl],
                              preferred_element_type=jnp.float32)
        cks.append(ck)
        cvs.append(cv)
    ck_ref[0] = jnp.concatenate(cks, axis=-1)
    cv_ref[0] = jnp.concatenate(cvs, axis=-1)


def _k2(qkv3, wkc, wvc, bkc, bvc, kpos, vpos):
    cd = CBS * DH
    b, n, _ = qkv3.shape
    nb = n // CBS
    q4 = qkv3.reshape(b, nb, CBS * 3 * INNER)
    kspec = [pl.BlockSpec((1, nb, 2 * DH),
                          (lambda bi, j, jr_=jr: (bi, 0, 18 * jr_ + NPAIR + j)))
             for jr in range(CBS)]
    vspec = [pl.BlockSpec((1, nb, 2 * DH),
                          (lambda bi, j, jr_=jr: (bi, 0, 18 * jr_ + 2 * NPAIR + j)))
             for jr in range(CBS)]
    return pl.pallas_call(
        _k2_body,
        grid=(b, NPAIR),
        in_specs=kspec + vspec + [
            pl.BlockSpec((cd, DH), lambda bi, j: (0, 0)),
            pl.BlockSpec((cd, DH), lambda bi, j: (0, 0)),
            pl.BlockSpec((1, DH), lambda bi, j: (0, 0)),
            pl.BlockSpec((1, DH), lambda bi, j: (0, 0)),
            pl.BlockSpec((1, cd), lambda bi, j: (0, 0)),
            pl.BlockSpec((1, cd), lambda bi, j: (0, 0)),
        ],
        out_specs=[
            pl.BlockSpec((1, nb, 2 * DH), lambda bi, j: (bi, 0, j)),
            pl.BlockSpec((1, nb, 2 * DH), lambda bi, j: (bi, 0, j)),
        ],
        out_shape=[
            jax.ShapeDtypeStruct((b, nb, INNER), jnp.float32),
            jax.ShapeDtypeStruct((b, nb, INNER), jnp.float32),
        ],
    )(q4, q4, q4, q4, q4, q4, q4, q4, wkc, wvc, bkc[None], bvc[None],
      kpos.reshape(1, cd), vpos.reshape(1, cd))


def _k3_body(q_ref, ck_ref, cv_ref, mk_ref, mv_ref, cout_ref, sel_ref,
             imp_ref, bias_ref, *, tq, nb, qt):
    j = pl.program_id(1)
    q2 = q_ref[0]
    ck2 = ck_ref[0]
    cv2 = cv_ref[0]
    mk2 = mk_ref[0]
    mv2 = mv_ref[0]

    @pl.when(j == 0)
    def _():
        ivec = qt * tq + jax.lax.broadcasted_iota(jnp.int32, (tq, nb), 0)
        mvec = jax.lax.broadcasted_iota(jnp.int32, (tq, nb), 1)
        bias_ref[...] = jnp.where((CBS * mvec + (CBS - 1)) <= ivec, 0.0, NEG)

    bias = bias_ref[...]
    couts = []
    imps = []
    for hh in range(2):
        q = q2[:, hh * DH:(hh + 1) * DH]
        ck = ck2[:, hh * DH:(hh + 1) * DH]
        cv = cv2[:, hh * DH:(hh + 1) * DH]
        s = jax.lax.dot_general(q, ck, (((1,), (1,)), ((), ())),
                                preferred_element_type=jnp.float32) * SCALE
        s = s + bias
        smem = jnp.sum(q * mk2[:, hh * DH:(hh + 1) * DH], -1,
                       keepdims=True) * SCALE
        mx = jnp.maximum(jnp.max(s, -1, keepdims=True), smem)
        eb = jnp.exp(s - mx)
        em = jnp.exp(smem - mx)
        den = jnp.sum(eb, -1, keepdims=True) + em
        rc = 1.0 / den
        couts.append((jnp.dot(eb, cv, preferred_element_type=jnp.float32)
                      + em * mv2[:, hh * DH:(hh + 1) * DH]) * rc)
        imps.append(eb * rc)
    cout_ref[0] = jnp.concatenate(couts, axis=-1).astype(jnp.bfloat16)
    impc = (imps[0] + imps[1]) * (1.0 / HEADS)

    @pl.when(j == 0)
    def _():
        imp_ref[...] = impc

    @pl.when(j != 0)
    def _():
        imp_ref[...] = imp_ref[...] + impc

    @pl.when(j == NPAIR - 1)
    def _():
        vimp = imp_ref[...]
        jj = jax.lax.broadcasted_iota(jnp.int32, (tq, nb), 1)
        m1 = jnp.max(vimp, -1, keepdims=True)
        i1 = jnp.min(jnp.where(vimp == m1, jj, nb), -1, keepdims=True)
        v2 = jnp.where(jj == i1, -jnp.inf, vimp)
        m2 = jnp.max(v2, -1, keepdims=True)
        i2 = jnp.min(jnp.where(v2 == m2, jj, nb), -1, keepdims=True)
        sel_ref[0] = jnp.concatenate([i1, i2], axis=-1)


def _k3(qkv3, ck, cv, mem_kp, mem_vp, qt, tq=512):
    b = qkv3.shape[0]
    nb = (qt + 1) * tq // CBS

    def body(*refs):
        _k3_body(*refs, tq=tq, nb=nb, qt=qt)

    return pl.pallas_call(
        body,
        grid=(b, NPAIR),
        in_specs=[
            pl.BlockSpec((1, tq, 2 * DH), lambda bi, j: (bi, qt, j)),
            pl.BlockSpec((1, nb, 2 * DH), lambda bi, j: (bi, 0, j)),
            pl.BlockSpec((1, nb, 2 * DH), lambda bi, j: (bi, 0, j)),
            pl.BlockSpec((1, 1, 2 * DH), lambda bi, j: (j, 0, 0)),
            pl.BlockSpec((1, 1, 2 * DH), lambda bi, j: (j, 0, 0)),
        ],
        out_specs=[
            pl.BlockSpec((1, tq, 2 * DH), lambda bi, j: (bi, 0, j)),
            pl.BlockSpec((1, tq, NSEL), lambda bi, j: (bi, 0, 0)),
        ],
        out_shape=[
            jax.ShapeDtypeStruct((b, tq, INNER), jnp.bfloat16),
            jax.ShapeDtypeStruct((b, tq, NSEL), jnp.int32),
        ],
        scratch_shapes=[
            pltpu.VMEM((tq, nb), jnp.float32),
            pltpu.VMEM((tq, nb), jnp.float32),
        ],
    )(qkv3, ck, cv, mem_kp, mem_vp)


def _k5_body(q_ref, k_ref, v_ref, cout_ref, sel_ref, g_ref, prev_ref,
             out_ref, wc_ref, *, tq, nw, qt):
    j = pl.program_id(1)
    qs = qt * tq

    @pl.when(j == 0)
    def _():
        i2 = qs + jax.lax.broadcasted_iota(jnp.int32, (tq, nw), 0)
        j2 = jax.lax.broadcasted_iota(jnp.int32, (tq, nw), 1)
        jblk = j2 // SBS
        sel = sel_ref[0]
        sel0 = sel[:, 0:1]
        sel1 = sel[:, 1:2]
        icol0 = qs + jax.lax.broadcasted_iota(jnp.int32, (tq, 1), 0)
        own = icol0 // SBS
        w = ((jblk == sel0).astype(jnp.float32)
             + (jblk == sel1).astype(jnp.float32)
             + (jblk == own).astype(jnp.float32))
        causal = j2 <= i2
        wc_ref[...] = jnp.where(causal, w, 0.0)

    q2 = q_ref[0]
    k2 = k_ref[0]
    v2 = v_ref[0]
    icol = qs + jax.lax.broadcasted_iota(jnp.int32, (tq, 1), 0)
    kti = k_ref[0, pl.ds(qs, tq), :].astype(jnp.float32)
    vti = v_ref[0, pl.ds(qs, tq), :].astype(jnp.float32)
    pstart = jnp.maximum(qs - 1, 0)
    kp0 = k_ref[0, pl.ds(pstart, 1), :].astype(jnp.float32)
    vp0 = v_ref[0, pl.ds(pstart, 1), :].astype(jnp.float32)
    kpi = jnp.concatenate([kp0, kti[:-1]], axis=0)
    vpi = jnp.concatenate([vp0, vti[:-1]], axis=0)
    wcm = wc_ref[...]
    outs = []
    for hh in range(2):
        sl = slice(hh * DH, (hh + 1) * DH)
        q = q2[:, sl]
        k = k2[:, sl]
        v = v2[:, sl]
        s = jax.lax.dot_general(q, k, (((1,), (1,)), ((), ())),
                                preferred_element_type=jnp.float32) * SCALE
        mx = jnp.max(s, -1, keepdims=True)
        e = wcm * jnp.exp(s - mx)
        den = jnp.sum(e, -1, keepdims=True)
        sout = jnp.dot(e.astype(jnp.bfloat16), v,
                       preferred_element_type=jnp.float32) * (1.0 / den)
        qf = q.astype(jnp.float32)
        kt = kti[:, sl]
        vt = vti[:, sl]
        kprev = kpi[:, sl]
        vprev = vpi[:, sl]
        s1 = jnp.sum(qf * kt, -1, keepdims=True) * SCALE
        s0 = jnp.sum(qf * kprev, -1, keepdims=True) * SCALE
        s0 = jnp.where(icol > 0, s0, NEG)
        mw = jnp.maximum(s0, s1)
        e0 = jnp.exp(s0 - mw)
        e1 = jnp.exp(s1 - mw)
        wout = (e0 * vprev + e1 * vt) * (1.0 / (e0 + e1))
        liota = jax.lax.broadcasted_iota(jnp.int32, (3 * HEADS, 3), 0)
        giota = jax.lax.broadcasted_iota(jnp.int32, (3 * HEADS, 3), 1)
        eh = (liota == giota * HEADS + (2 * j + hh)).astype(jnp.float32)
        g = jnp.dot(g_ref[0], eh, preferred_element_type=jnp.float32)
        outs.append(g[:, 0:1] * cout_ref[0][:, sl] + g[:, 1:2] * sout
                    + g[:, 2:3] * wout)
    out_ref[0] = jnp.concatenate(outs, axis=-1)


def _k5(qkvb3, cout, sel, g36_3, prev, qt, tq=512):
    b, n, _ = qkvb3.shape
    nw = (qt + 1) * tq

    def body(*refs):
        _k5_body(*refs, tq=tq, nw=nw, qt=qt)

    return pl.pallas_call(
        body,
        grid=(b, NPAIR),
        in_specs=[
            pl.BlockSpec((1, tq, 2 * DH), lambda bi, j: (bi, qt, j)),
            pl.BlockSpec((1, nw, 2 * DH), lambda bi, j: (bi, 0, NPAIR + j)),
            pl.BlockSpec((1, nw, 2 * DH), lambda bi, j: (bi, 0, 2 * NPAIR + j)),
            pl.BlockSpec((1, tq, 2 * DH), lambda bi, j: (bi, 0, j)),
            pl.BlockSpec((1, tq, NSEL), lambda bi, j: (bi, 0, 0)),
            pl.BlockSpec((1, tq, 3 * HEADS), lambda bi, j: (bi, qt, 0)),
            pl.BlockSpec((1, 8, 2 * DH), lambda bi, j: (bi, 0, j)),
        ],
        out_specs=pl.BlockSpec((1, tq, 2 * DH), lambda bi, j: (bi, qt, j)),
        out_shape=jax.ShapeDtypeStruct((b, n, INNER), jnp.float32),
        scratch_shapes=[pltpu.VMEM((tq, nw), jnp.float32)],
        input_output_aliases={6: 0},
    )(qkvb3, qkvb3, qkvb3, cout, sel, g36_3, prev)


def _k67_body(a_ref, wo_ref, r_ref, g2_ref, b2_ref, w1_ref, b1_ref,
              y_ref, h_ref):
    y = (jnp.dot(a_ref[...].astype(jnp.bfloat16), wo_ref[...],
                 preferred_element_type=jnp.float32) + r_ref[...])
    y_ref[...] = y
    m = jnp.mean(y, -1, keepdims=True)
    v = jnp.mean((y - m) ** 2, -1, keepdims=True)
    xn = (y - m) / jnp.sqrt(v + 1e-5) * g2_ref[...] + b2_ref[...]
    z = jnp.dot(xn.astype(jnp.bfloat16), w1_ref[...],
                preferred_element_type=jnp.float32) + b1_ref[...]
    h_ref[...] = jnp.where(z >= 0, z, 0.01 * z).astype(jnp.bfloat16)


def _k67(a, wo, res, ln_g, ln_b, w1, b1, tr=1024):
    r, d_in = a.shape

    return pl.pallas_call(
        _k67_body,
        grid=(r // tr,),
        in_specs=[
            pl.BlockSpec((tr, d_in), lambda i: (i, 0)),
            pl.BlockSpec((d_in, DIM), lambda i: (0, 0)),
            pl.BlockSpec((tr, DIM), lambda i: (i, 0)),
            pl.BlockSpec((1, DIM), lambda i: (0, 0)),
            pl.BlockSpec((1, DIM), lambda i: (0, 0)),
            pl.BlockSpec((DIM, MLP_D), lambda i: (0, 0)),
            pl.BlockSpec((1, MLP_D), lambda i: (0, 0)),
        ],
        out_specs=[
            pl.BlockSpec((tr, DIM), lambda i: (i, 0)),
            pl.BlockSpec((tr, MLP_D), lambda i: (i, 0)),
        ],
        out_shape=[
            jax.ShapeDtypeStruct((r, DIM), jnp.float32),
            jax.ShapeDtypeStruct((r, MLP_D), jnp.bfloat16),
        ],
    )(a, wo, res, ln_g[None], ln_b[None], w1, b1[None])


def _k8_body(a_ref, w_ref, b_ref, r_ref, o_ref):
    o_ref[...] = (jnp.dot(a_ref[...], w_ref[...],
                          preferred_element_type=jnp.float32)
                  + b_ref[...] + r_ref[...])


def _k8(a, w, bias, res, tr=1024):
    r, d_in = a.shape
    d_out = w.shape[1]
    return pl.pallas_call(
        _k8_body,
        grid=(r // tr,),
        in_specs=[
            pl.BlockSpec((tr, d_in), lambda i: (i, 0)),
            pl.BlockSpec((d_in, d_out), lambda i: (0, 0)),
            pl.BlockSpec((1, d_out), lambda i: (0, 0)),
            pl.BlockSpec((tr, d_out), lambda i: (i, 0)),
        ],
        out_specs=pl.BlockSpec((tr, d_out), lambda i: (i, 0)),
        out_shape=jax.ShapeDtypeStruct((r, d_out), jnp.float32),
    )(a, w, bias[None], res)


def _k81_body(a_ref, w2_ref, b2_ref, r_ref, g1_ref, b1_ref,
              wqkv_ref, wg_ref, bg_ref, x_ref, qkv_ref, qkvb_ref, g_ref):
    x = (jnp.dot(a_ref[...], w2_ref[...], preferred_element_type=jnp.float32)
         + b2_ref[...] + r_ref[...])
    x_ref[...] = x
    m = jnp.mean(x, -1, keepdims=True)
    v = jnp.mean((x - m) ** 2, -1, keepdims=True)
    xn = (x - m) / jnp.sqrt(v + 1e-5) * g1_ref[...] + b1_ref[...]
    qv = jnp.dot(xn, wqkv_ref[...], preferred_element_type=jnp.float32)
    qkv_ref[...] = qv
    qkvb_ref[...] = qv.astype(jnp.bfloat16)
    g_ref[...] = jax.nn.sigmoid(
        jnp.dot(xn, wg_ref[...], preferred_element_type=jnp.float32) + bg_ref[...])


def _k81(a, w2, b2, res, ln_g, ln_b, wqkv, wg, bg, tr=512):
    r, d_in = a.shape
    return pl.pallas_call(
        _k81_body,
        grid=(r // tr,),
        in_specs=[
            pl.BlockSpec((tr, d_in), lambda i: (i, 0)),
            pl.BlockSpec((d_in, DIM), lambda i: (0, 0)),
            pl.BlockSpec((1, DIM), lambda i: (0, 0)),
            pl.BlockSpec((tr, DIM), lambda i: (i, 0)),
            pl.BlockSpec((1, DIM), lambda i: (0, 0)),
            pl.BlockSpec((1, DIM), lambda i: (0, 0)),
            pl.BlockSpec((DIM, 3 * INNER), lambda i: (0, 0)),
            pl.BlockSpec((DIM, 3 * HEADS), lambda i: (0, 0)),
            pl.BlockSpec((1, 3 * HEADS), lambda i: (0, 0)),
        ],
        out_specs=[
            pl.BlockSpec((tr, DIM), lambda i: (i, 0)),
            pl.BlockSpec((tr, 3 * INNER), lambda i: (i, 0)),
            pl.BlockSpec((tr, 3 * INNER), lambda i: (i, 0)),
            pl.BlockSpec((tr, 3 * HEADS), lambda i: (i, 0)),
        ],
        out_shape=[
            jax.ShapeDtypeStruct((r, DIM), jnp.float32),
            jax.ShapeDtypeStruct((r, 3 * INNER), jnp.float32),
            jax.ShapeDtypeStruct((r, 3 * INNER), jnp.bfloat16),
            jax.ShapeDtypeStruct((r, 3 * HEADS), jnp.float32),
        ],
    )(a, w2, b2[None], res, ln_g[None], ln_b[None], wqkv, wg, bg[None])


def _attn(qkv, qkvb, g36, p, b, n):
    qkv3 = qkv.reshape(b, n, 3 * INNER)
    qkvb3 = qkvb.reshape(b, n, 3 * INNER)
    ck, cv = _k2(qkv3, p['Wkc'], p['Wvc'], p['bkc'], p['bvc'],
                 p['k_pos'], p['v_pos'])
    g36_3 = g36.reshape(b, n, 3 * HEADS)
    mem_kp = p['mem_k'].reshape(NPAIR, 1, 2 * DH)
    mem_vp = p['mem_v'].reshape(NPAIR, 1, 2 * DH)
    tq = 512
    comb = jnp.zeros((b, n, INNER), jnp.float32)
    for qt in range(n // tq):
        cout_t, sel_t = _k3(qkv3, ck, cv, mem_kp, mem_vp, qt, tq=tq)
        comb = _k5(qkvb3, cout_t, sel_t, g36_3, comb, qt, tq=tq)
    return comb.reshape(b * n, INNER)


def kernel(x, params):
    b, n, _ = x.shape
    p0, p1 = params[0], params[1]
    x2 = x.reshape(b * n, DIM)
    qkv, qkvb, g36 = _k1(x2, p0['ln1_g'], p0['ln1_b'], p0['Wqkv'], p0['Wg'],
                         p0['bg'])
    comb2 = _attn(qkv, qkvb, g36, p0, b, n)
    y0, h0 = _k67(comb2, p0['Wo'].astype(jnp.bfloat16), x2,
                  p0['ln2_g'], p0['ln2_b'],
                  p0['W1'].astype(jnp.bfloat16), p0['b1'])
    x1, qkv1, qkvb1, g36_1 = _k81(h0, p0['W2'].astype(jnp.bfloat16), p0['b2'],
                                  y0, p1['ln1_g'], p1['ln1_b'], p1['Wqkv'],
                                  p1['Wg'], p1['bg'])
    comb2_1 = _attn(qkv1, qkvb1, g36_1, p1, b, n)
    y1, h1 = _k67(comb2_1, p1['Wo'].astype(jnp.bfloat16), x1,
                  p1['ln2_g'], p1['ln2_b'],
                  p1['W1'].astype(jnp.bfloat16), p1['b1'])
    out = _k8(h1, p1['W2'].astype(jnp.bfloat16), p1['b2'], y1)
    return out.reshape(b, n, DIM)
